# Initial kernel scaffold; baseline (speedup 1.0000x reference)
import jax
import jax.numpy as jnp
from jax.experimental import pallas as pl


def kernel(rec_h, rec_x, rec_e_index, rec_e_type, rec_batch, lig_h_type, lig_x, lig_e_index, lig_e_type, lig_batch, timestep, inter_e_index, inter_e_type, params):
    raise NotImplementedError("write your pallas kernel here")


# speedup vs baseline: 1.1703x; 1.1703x over previous
import functools

import jax
import jax.numpy as jnp
import numpy as np
from jax import lax
from jax.experimental import pallas as pl
from jax.experimental.pallas import tpu as pltpu
from jax.experimental.pallas import tpu_sc as plsc

N_REC = 10000
N_LIG = 10000
H = 128
EE = 64
HID = 128
TD = 128
NT = 1000
XP = 16

_EBLK = 1000
_NBLK = 1000

_INTERP = False


def _pe_table_np(d, n):
    pos = np.arange(n)[:, None].astype(np.float64)
    i = np.arange(d)[None, :]
    ang = pos / np.power(10000.0, (2 * (i // 2)) / d)
    t = np.zeros((n, d))
    t[:, 0::2] = np.sin(ang[:, 0::2])
    t[:, 1::2] = np.cos(ang[:, 1::2])
    return jnp.asarray(t, dtype=jnp.float32)


_PE = _pe_table_np(TD, NT)


def _ln(x):
    m = jnp.mean(x, axis=-1, keepdims=True)
    v = jnp.mean((x - m) * (x - m), axis=-1, keepdims=True)
    return (x - m) * lax.rsqrt(v + 1e-5)


def _silu_ln(x):
    y = _ln(x)
    return y * jax.nn.sigmoid(y)


def _fs(shape):
    nd = len(shape)
    return pl.BlockSpec(shape, lambda i, _n=nd: (0,) * _n)


def _rowspec(blk, width):
    return pl.BlockSpec((blk, width), lambda i: (i, 0))


def _dot(a, b):
    return jnp.dot(a, b, preferred_element_type=jnp.float32)


def _msg_body(has_x, hs_ref, hd_ref, e_ref, xs_ref, xd_ref,
              w1hs, w1hd, w1e, w1d, b1, w2, b2, we, be, *rest):
    if has_x:
        wx, bx, m_out, e_out, sc_out = rest
    else:
        m_out, e_out = rest
    hs = hs_ref[...]
    hd = hd_ref[...]
    e = e_ref[...]
    diff = xd_ref[...] - xs_ref[...]
    d2 = jnp.sum(diff * diff, axis=1, keepdims=True)
    z = (_dot(hs, w1hs[...]) + _dot(hd, w1hd[...]) + _dot(e, w1e[...])
         + d2 * w1d[...] + b1[...])
    z = _silu_ln(z)
    z = _dot(z, w2[...]) + b2[...]
    m = _silu_ln(z)
    m_out[...] = m
    e_out[...] = e + _dot(m, we[...]) + be[...]
    if has_x:
        coef = jnp.sum(m * wx[...], axis=1, keepdims=True) + bx[...]
        sc = diff * coef
        col = lax.broadcasted_iota(jnp.int32, sc.shape, 1)
        sc_out[...] = jnp.where(col == 3, 1.0, sc)


def _msg_call(hs, hd, e, xs, xd, msg_ps, e_ps, x_ps=None):
    E = hs.shape[0]
    has_x = x_ps is not None
    w1 = msg_ps[0]['w']
    w1hs = w1[:H]
    w1hd = w1[H:2 * H]
    w1e = w1[2 * H:2 * H + EE]
    w1d = w1[2 * H + EE:].reshape(1, HID)
    b1 = msg_ps[0]['b'].reshape(1, HID)
    w2 = msg_ps[1]['w']
    b2 = msg_ps[1]['b'].reshape(1, HID)
    we = e_ps['w']
    be = e_ps['b'].reshape(1, EE)
    args = [hs, hd, e, xs, xd, w1hs, w1hd, w1e, w1d, b1, w2, b2, we, be]
    outs = [jax.ShapeDtypeStruct((E, HID), jnp.float32),
            jax.ShapeDtypeStruct((E, EE), jnp.float32)]
    out_specs = [_rowspec(_EBLK, HID), _rowspec(_EBLK, EE)]
    if has_x:
        args += [x_ps['w'].reshape(1, HID), x_ps['b'].reshape(1, 1)]
        outs.append(jax.ShapeDtypeStruct((E, XP), jnp.float32))
        out_specs.append(_rowspec(_EBLK, XP))
    in_specs = [_rowspec(_EBLK, H), _rowspec(_EBLK, H), _rowspec(_EBLK, EE),
                _rowspec(_EBLK, XP), _rowspec(_EBLK, XP)]
    in_specs += [_fs(a.shape) for a in args[5:]]
    return pl.pallas_call(
        functools.partial(_msg_body, has_x),
        grid=(E // _EBLK,),
        in_specs=in_specs,
        out_specs=out_specs,
        out_shape=outs,
        interpret=_INTERP,
    )(*args)


def _onehot_body(nvoc, t_ref, emb, out):
    oh = (lax.broadcasted_iota(jnp.int32, (t_ref.shape[0], nvoc), 1)
          == t_ref[...]).astype(jnp.float32)
    out[...] = _dot(oh, emb[...])


def _onehot_embed(types, emb):
    E = types.shape[0]
    nvoc = emb.shape[0]
    if nvoc % 8 != 0:
        emb = jnp.pad(emb, ((0, 8 - nvoc % 8), (0, 0)))
        nvoc = emb.shape[0]
    d = emb.shape[1]
    return pl.pallas_call(
        functools.partial(_onehot_body, nvoc),
        grid=(E // _EBLK,),
        in_specs=[_rowspec(_EBLK, 1), _fs(emb.shape)],
        out_specs=_rowspec(_EBLK, d),
        out_shape=jax.ShapeDtypeStruct((E, d), jnp.float32),
        interpret=_INTERP,
    )(types.reshape(E, 1), emb)


def _merge_e_body(e_ref, t_ref, wa, wb, b1, w2, b2, out):
    z = _dot(e_ref[...], wa[...]) + _dot(t_ref[...], wb[...]) + b1[...]
    z = _silu_ln(z)
    z = _dot(z, w2[...]) + b2[...]
    out[...] = _silu_ln(z)


def _merge_e_call(e, t, ps):
    E = e.shape[0]
    w1 = ps[0]['w']
    args = [e, t, w1[:EE], w1[EE:], ps[0]['b'].reshape(1, EE),
            ps[1]['w'], ps[1]['b'].reshape(1, EE)]
    return pl.pallas_call(
        _merge_e_body,
        grid=(E // _EBLK,),
        in_specs=[_rowspec(_EBLK, EE), _rowspec(_EBLK, TD)]
        + [_fs(a.shape) for a in args[2:]],
        out_specs=_rowspec(_EBLK, EE),
        out_shape=jax.ShapeDtypeStruct((E, EE), jnp.float32),
        interpret=_INTERP,
    )(*args)


def _rec_init_body(h_ref, w_ref, out):
    out[...] = _dot(h_ref[...], w_ref[...])


def _rec_init_call(rec_h, w):
    return pl.pallas_call(
        _rec_init_body,
        grid=(N_REC // _NBLK,),
        in_specs=[_rowspec(_NBLK, rec_h.shape[1]), _fs(w.shape)],
        out_specs=_rowspec(_NBLK, H),
        out_shape=jax.ShapeDtypeStruct((N_REC, H), jnp.float32),
        interpret=_INTERP,
    )(rec_h, w)


def _lig_init_body(ty_ref, ba_ref, t_ref, emb, wa, wb, b1, w2, b2,
                   h_out, tb_out):
    blk = ty_ref.shape[0]
    oh = (lax.broadcasted_iota(jnp.int32, (blk, 32), 1)
          == ty_ref[...]).astype(jnp.float32)
    h0 = _dot(oh, emb[...])
    ohb = (lax.broadcasted_iota(jnp.int32, (blk, 8), 1)
           == ba_ref[...]).astype(jnp.float32)
    tb = _dot(ohb, t_ref[...])
    tb_out[...] = tb
    z = _dot(h0, wa[...]) + _dot(tb, wb[...]) + b1[...]
    z = _silu_ln(z)
    z = _dot(z, w2[...]) + b2[...]
    h_out[...] = _silu_ln(z)


def _lig_init_call(lig_h_type, lig_batch, t, emb, ps):
    w1 = ps[0]['w']
    args = [lig_h_type.reshape(N_LIG, 1), lig_batch.reshape(N_LIG, 1), t,
            emb, w1[:H], w1[H:], ps[0]['b'].reshape(1, H),
            ps[1]['w'], ps[1]['b'].reshape(1, H)]
    return pl.pallas_call(
        _lig_init_body,
        grid=(N_LIG // _NBLK,),
        in_specs=[_rowspec(_NBLK, 1), _rowspec(_NBLK, 1)]
        + [_fs(a.shape) for a in args[2:]],
        out_specs=[_rowspec(_NBLK, H), _rowspec(_NBLK, TD)],
        out_shape=[jax.ShapeDtypeStruct((N_LIG, H), jnp.float32),
                   jax.ShapeDtypeStruct((N_LIG, TD), jnp.float32)],
        interpret=_INTERP,
    )(*args)


def _upd_body(nacc, h_ref, wa, wb, b1, w2, b2, *rest):
    accs, out = rest[:nacc], rest[nacc]
    h = h_ref[...]
    agg = accs[0][...][:, :H]
    for a in accs[1:]:
        agg = agg + a[...][:, :H]
    z = _dot(h, wa[...]) + _dot(agg, wb[...]) + b1[...]
    z = _silu_ln(z)
    z = _dot(z, w2[...]) + b2[...]
    out[...] = h + z


def _upd_call(h, accs, ps):
    n = h.shape[0]
    w1 = ps[0]['w']
    flat = []
    for acc in accs:
        for p in range(acc.shape[0]):
            flat.append(acc[p])
    args = [h, w1[:H], w1[H:], ps[0]['b'].reshape(1, HID),
            ps[1]['w'], ps[1]['b'].reshape(1, H)] + flat
    return pl.pallas_call(
        functools.partial(_upd_body, len(flat)),
        grid=(n // _NBLK,),
        in_specs=[_rowspec(_NBLK, H)] + [_fs(a.shape) for a in args[1:6]]
        + [_rowspec(_NBLK, f.shape[1]) for f in flat],
        out_specs=_rowspec(_NBLK, H),
        out_shape=jax.ShapeDtypeStruct((n, H), jnp.float32),
        interpret=_INTERP,
    )(*args)


def _xupd_body(x_ref, *rest):
    accs, out = rest[:-1], rest[-1]
    acc = accs[0][...]
    for a in accs[1:]:
        acc = acc + a[...]
    num = acc[:, HID:HID + XP]
    cnt = acc[:, HID + 3:HID + 4]
    col = lax.broadcasted_iota(jnp.int32, num.shape, 1)
    num = jnp.where(col == 3, 0.0, num)
    out[...] = x_ref[...] + num / (cnt + 1e-8)


def _xupd_call(x, acc):
    n = x.shape[0]
    flat = [acc[p] for p in range(acc.shape[0])]
    return pl.pallas_call(
        _xupd_body,
        grid=(n // _NBLK,),
        in_specs=[_rowspec(_NBLK, XP)]
        + [_rowspec(_NBLK, f.shape[1]) for f in flat],
        out_specs=_rowspec(_NBLK, XP),
        out_shape=jax.ShapeDtypeStruct((n, XP), jnp.float32),
        interpret=_INTERP,
    )(x, *flat)


def _readout_body(h_ref, w1, b1, w2, b2, w3, b3, out):
    z = _dot(h_ref[...], w1[...]) + b1[...]
    z = _silu_ln(z)
    z = _dot(z, w2[...]) + b2[...]
    z = _silu_ln(z)
    out[...] = _dot(z, w3[...]) + b3[...]


def _readout_call(h, ps):
    n = h.shape[0]
    d = ps[2]['w'].shape[1]
    args = [h, ps[0]['w'], ps[0]['b'].reshape(1, -1),
            ps[1]['w'], ps[1]['b'].reshape(1, -1),
            ps[2]['w'], ps[2]['b'].reshape(1, -1)]
    return pl.pallas_call(
        _readout_body,
        grid=(n // _NBLK,),
        in_specs=[_rowspec(_NBLK, H)] + [_fs(a.shape) for a in args[1:]],
        out_specs=_rowspec(_NBLK, d),
        out_shape=jax.ShapeDtypeStruct((n, d), jnp.float32),
        interpret=_INTERP,
    )(*args)


def _gather_rows(table, idx):
    return jnp.take(table, idx, axis=0)


def _scatter_add(vals, idx, n):
    return jax.ops.segment_sum(vals, idx, num_segments=n)[None]


def kernel(rec_h, rec_x, rec_e_index, rec_e_type, rec_batch, lig_h_type,
           lig_x, lig_e_index, lig_e_type, lig_batch, timestep,
           inter_e_index, inter_e_type, params):
    t = _PE[timestep]
    for i, p in enumerate(params['embd_timestep']):
        t = t @ p['w'] + p['b']
        t = jax.nn.silu(_ln(t))

    rsf = jnp.concatenate([rec_e_index[0], rec_e_index[1]])
    rdf = jnp.concatenate([rec_e_index[1], rec_e_index[0]])
    lsf = jnp.concatenate([lig_e_index[0], lig_e_index[1]])
    ldf = jnp.concatenate([lig_e_index[1], lig_e_index[0]])
    isrc, idst = inter_e_index[0], inter_e_index[1]

    rec_xp = jnp.pad(rec_x, ((0, 0), (0, XP - 3)))
    lig_xp = jnp.pad(lig_x, ((0, 0), (0, XP - 3)))

    rec_h_cur = _rec_init_call(rec_h, params['embd_rec_h'])
    lig_h, tB = _lig_init_call(lig_h_type, lig_batch, t,
                               params['embd_lig_h'], params['merge_lig_h'])

    full_rec_e = _onehot_embed(jnp.concatenate([rec_e_type, rec_e_type]),
                               params['embd_rec_e'])
    lig_e_raw = _onehot_embed(jnp.concatenate([lig_e_type, lig_e_type]),
                              params['embd_lig_e'])
    inter_e_raw = _onehot_embed(inter_e_type, params['embd_inter_e'])
    t_lig_e = _gather_rows(tB, ldf)
    t_inter_e = _gather_rows(tB, idst)
    full_lig_e = _merge_e_call(lig_e_raw, t_lig_e, params['merge_lig_e'])
    inter_e = _merge_e_call(inter_e_raw, t_inter_e, params['merge_inter_e'])

    gx_rs = _gather_rows(rec_xp, rsf)
    gx_rd = _gather_rows(rec_xp, rdf)
    gx_is = _gather_rows(rec_xp, isrc)

    for lp in params['layers']:
        ghs = _gather_rows(rec_h_cur, rsf)
        ghd = _gather_rows(rec_h_cur, rdf)
        m_rr, full_rec_e = _msg_call(ghs, ghd, full_rec_e, gx_rs, gx_rd,
                                     lp['rr_msg'], lp['rr_e'])
        acc_rr = _scatter_add(m_rr, rdf, N_REC)
        rec_h_cur = _upd_call(rec_h_cur, [acc_rr], lp['rr_upd'])

        ghs = _gather_rows(lig_h, lsf)
        ghd = _gather_rows(lig_h, ldf)
        gxs = _gather_rows(lig_xp, lsf)
        gxd = _gather_rows(lig_xp, ldf)
        m_ll, full_lig_e, sc_ll = _msg_call(ghs, ghd, full_lig_e, gxs, gxd,
                                            lp['ll_msg'], lp['ll_e'],
                                            lp['ll_x'])
        acc_ll = _scatter_add(jnp.concatenate([m_ll, sc_ll], axis=1), ldf,
                              N_LIG)
        lig_xp = _xupd_call(lig_xp, acc_ll)

        ghr = _gather_rows(rec_h_cur, isrc)
        ghl = _gather_rows(lig_h, idst)
        gxl = _gather_rows(lig_xp, idst)
        m_rl, inter_e, sc_rl = _msg_call(ghr, ghl, inter_e, gx_is, gxl,
                                         lp['rl_msg'], lp['rl_e'],
                                         lp['rl_x'])
        acc_rl = _scatter_add(jnp.concatenate([m_rl, sc_rl], axis=1), idst,
                              N_LIG)
        lig_xp = _xupd_call(lig_xp, acc_rl)
        lig_h = _upd_call(lig_h, [acc_ll, acc_rl], lp['ll_upd'])

    lig_h_out = _readout_call(lig_h, params['readout_lig_h'])
    return lig_h_out, lig_xp[:, :3]


# speedup vs baseline: 2.4712x; 2.1115x over previous
import functools

import jax
import jax.numpy as jnp
import numpy as np
from jax import lax
from jax.experimental import pallas as pl
from jax.experimental.pallas import tpu as pltpu
from jax.experimental.pallas import tpu_sc as plsc

N_REC = 10000
N_LIG = 10000
H = 128
EE = 64
HID = 128
TD = 128
NT = 1000
XP = 16

_EBLK = 1000
_NBLK = 1000

_INTERP = False


def _pe_table_np(d, n):
    pos = np.arange(n)[:, None].astype(np.float64)
    i = np.arange(d)[None, :]
    ang = pos / np.power(10000.0, (2 * (i // 2)) / d)
    t = np.zeros((n, d))
    t[:, 0::2] = np.sin(ang[:, 0::2])
    t[:, 1::2] = np.cos(ang[:, 1::2])
    return t.astype(np.float32)


_PE = _pe_table_np(TD, NT)


def _ln(x):
    m = jnp.mean(x, axis=-1, keepdims=True)
    v = jnp.mean((x - m) * (x - m), axis=-1, keepdims=True)
    return (x - m) * lax.rsqrt(v + 1e-5)


def _silu_ln(x):
    y = _ln(x)
    return y * jax.nn.sigmoid(y)


def _fs(shape):
    nd = len(shape)
    return pl.BlockSpec(shape, lambda i, _n=nd: (0,) * _n)


def _rowspec(blk, width):
    return pl.BlockSpec((blk, width), lambda i: (i, 0))


def _dot(a, b):
    return jnp.dot(a, b, preferred_element_type=jnp.float32)


def _msg_body(has_x, gs_ref, gd_ref, e_ref,
              w1hs, w1hd, w1e, w1d, b1, w2, b2, we, be, *rest):
    if has_x:
        wx, bx, msc_out, e_out = rest
    else:
        msc_out, e_out = rest
    gs = gs_ref[...]
    gd = gd_ref[...]
    hs = gs[:, :H]
    hd = gd[:, :H]
    e = e_ref[...]
    diff = gd[:, H:] - gs[:, H:]
    d2 = jnp.sum(diff * diff, axis=1, keepdims=True)
    z = (_dot(hs, w1hs[...]) + _dot(hd, w1hd[...]) + _dot(e, w1e[...])
         + d2 * w1d[...] + b1[...])
    z = _silu_ln(z)
    z = _dot(z, w2[...]) + b2[...]
    m = _silu_ln(z)
    e_out[...] = e + _dot(m, we[...]) + be[...]
    if has_x:
        coef = jnp.sum(m * wx[...], axis=1, keepdims=True) + bx[...]
        sc = diff * coef
        col = lax.broadcasted_iota(jnp.int32, sc.shape, 1)
        sc = jnp.where(col == 3, 1.0, sc)
        msc_out[...] = jnp.concatenate([m, sc], axis=1)
    else:
        msc_out[...] = m


def _msg_call(gs, gd, e, msg_ps, e_ps, x_ps=None):
    E = gs.shape[0]
    has_x = x_ps is not None
    w1 = msg_ps[0]['w']
    w1hs = w1[:H]
    w1hd = w1[H:2 * H]
    w1e = w1[2 * H:2 * H + EE]
    w1d = w1[2 * H + EE:].reshape(1, HID)
    b1 = msg_ps[0]['b'].reshape(1, HID)
    w2 = msg_ps[1]['w']
    b2 = msg_ps[1]['b'].reshape(1, HID)
    we = e_ps['w']
    be = e_ps['b'].reshape(1, EE)
    args = [gs, gd, e, w1hs, w1hd, w1e, w1d, b1, w2, b2, we, be]
    mw = HID + XP if has_x else HID
    outs = [jax.ShapeDtypeStruct((E, mw), jnp.float32),
            jax.ShapeDtypeStruct((E, EE), jnp.float32)]
    out_specs = [_rowspec(_EBLK, mw), _rowspec(_EBLK, EE)]
    if has_x:
        args += [x_ps['w'].reshape(1, HID), x_ps['b'].reshape(1, 1)]
    in_specs = [_rowspec(_EBLK, H + XP), _rowspec(_EBLK, H + XP),
                _rowspec(_EBLK, EE)]
    in_specs += [_fs(a.shape) for a in args[3:]]
    return pl.pallas_call(
        functools.partial(_msg_body, has_x),
        grid=(E // _EBLK,),
        in_specs=in_specs,
        out_specs=out_specs,
        out_shape=outs,
        interpret=_INTERP,
    )(*args)


def _onehot_body(nvoc, t_ref, emb, out):
    oh = (lax.broadcasted_iota(jnp.int32, (t_ref.shape[0], nvoc), 1)
          == t_ref[...]).astype(jnp.float32)
    out[...] = _dot(oh, emb[...])


def _onehot_embed(types, emb):
    E = types.shape[0]
    nvoc = emb.shape[0]
    if nvoc % 8 != 0:
        emb = jnp.pad(emb, ((0, 8 - nvoc % 8), (0, 0)))
        nvoc = emb.shape[0]
    d = emb.shape[1]
    return pl.pallas_call(
        functools.partial(_onehot_body, nvoc),
        grid=(E // _EBLK,),
        in_specs=[_rowspec(_EBLK, 1), _fs(emb.shape)],
        out_specs=_rowspec(_EBLK, d),
        out_shape=jax.ShapeDtypeStruct((E, d), jnp.float32),
        interpret=_INTERP,
    )(types.reshape(E, 1), emb)


def _merge_e_body(e_ref, t_ref, wa, wb, b1, w2, b2, out):
    z = _dot(e_ref[...], wa[...]) + _dot(t_ref[...], wb[...]) + b1[...]
    z = _silu_ln(z)
    z = _dot(z, w2[...]) + b2[...]
    out[...] = _silu_ln(z)


def _merge_e_call(e, t, ps):
    E = e.shape[0]
    w1 = ps[0]['w']
    args = [e, t, w1[:EE], w1[EE:], ps[0]['b'].reshape(1, EE),
            ps[1]['w'], ps[1]['b'].reshape(1, EE)]
    return pl.pallas_call(
        _merge_e_body,
        grid=(E // _EBLK,),
        in_specs=[_rowspec(_EBLK, EE), _rowspec(_EBLK, TD)]
        + [_fs(a.shape) for a in args[2:]],
        out_specs=_rowspec(_EBLK, EE),
        out_shape=jax.ShapeDtypeStruct((E, EE), jnp.float32),
        interpret=_INTERP,
    )(*args)


def _rec_init_body(h_ref, w_ref, out):
    out[...] = _dot(h_ref[...], w_ref[...])


def _rec_init_call(rec_h, w):
    return pl.pallas_call(
        _rec_init_body,
        grid=(N_REC // _NBLK,),
        in_specs=[_rowspec(_NBLK, rec_h.shape[1]), _fs(w.shape)],
        out_specs=_rowspec(_NBLK, H),
        out_shape=jax.ShapeDtypeStruct((N_REC, H), jnp.float32),
        interpret=_INTERP,
    )(rec_h, w)


def _lig_init_body(ty_ref, ba_ref, t_ref, emb, wa, wb, b1, w2, b2,
                   h_out, tb_out):
    blk = ty_ref.shape[0]
    oh = (lax.broadcasted_iota(jnp.int32, (blk, 32), 1)
          == ty_ref[...]).astype(jnp.float32)
    h0 = _dot(oh, emb[...])
    ohb = (lax.broadcasted_iota(jnp.int32, (blk, 8), 1)
           == ba_ref[...]).astype(jnp.float32)
    tb = _dot(ohb, t_ref[...])
    tb_out[...] = tb
    z = _dot(h0, wa[...]) + _dot(tb, wb[...]) + b1[...]
    z = _silu_ln(z)
    z = _dot(z, w2[...]) + b2[...]
    h_out[...] = _silu_ln(z)


def _lig_init_call(lig_h_type, lig_batch, t, emb, ps):
    w1 = ps[0]['w']
    args = [lig_h_type.reshape(N_LIG, 1), lig_batch.reshape(N_LIG, 1), t,
            emb, w1[:H], w1[H:], ps[0]['b'].reshape(1, H),
            ps[1]['w'], ps[1]['b'].reshape(1, H)]
    return pl.pallas_call(
        _lig_init_body,
        grid=(N_LIG // _NBLK,),
        in_specs=[_rowspec(_NBLK, 1), _rowspec(_NBLK, 1)]
        + [_fs(a.shape) for a in args[2:]],
        out_specs=[_rowspec(_NBLK, H), _rowspec(_NBLK, TD)],
        out_shape=[jax.ShapeDtypeStruct((N_LIG, H), jnp.float32),
                   jax.ShapeDtypeStruct((N_LIG, TD), jnp.float32)],
        interpret=_INTERP,
    )(*args)


def _upd_body(nacc, h_ref, wa, wb, b1, w2, b2, *rest):
    accs, out = rest[:nacc], rest[nacc]
    h = h_ref[...]
    agg = accs[0][...][:, :H]
    for a in accs[1:]:
        agg = agg + a[...][:, :H]
    z = _dot(h, wa[...]) + _dot(agg, wb[...]) + b1[...]
    z = _silu_ln(z)
    z = _dot(z, w2[...]) + b2[...]
    out[...] = h + z


def _upd_call(h, accs, ps):
    n = h.shape[0]
    w1 = ps[0]['w']
    flat = []
    for acc in accs:
        for p in range(acc.shape[0]):
            flat.append(acc[p])
    args = [h, w1[:H], w1[H:], ps[0]['b'].reshape(1, HID),
            ps[1]['w'], ps[1]['b'].reshape(1, H)] + flat
    return pl.pallas_call(
        functools.partial(_upd_body, len(flat)),
        grid=(n // _NBLK,),
        in_specs=[_rowspec(_NBLK, H)] + [_fs(a.shape) for a in args[1:6]]
        + [_rowspec(_NBLK, f.shape[1]) for f in flat],
        out_specs=_rowspec(_NBLK, H),
        out_shape=jax.ShapeDtypeStruct((n, H), jnp.float32),
        interpret=_INTERP,
    )(*args)


def _xupd_body(x_ref, *rest):
    accs, out = rest[:-1], rest[-1]
    acc = accs[0][...]
    for a in accs[1:]:
        acc = acc + a[...]
    num = acc[:, HID:HID + XP]
    cnt = acc[:, HID + 3:HID + 4]
    col = lax.broadcasted_iota(jnp.int32, num.shape, 1)
    num = jnp.where(col == 3, 0.0, num)
    out[...] = x_ref[...] + num / (cnt + 1e-8)


def _xupd_call(x, acc):
    n = x.shape[0]
    flat = [acc[p] for p in range(acc.shape[0])]
    return pl.pallas_call(
        _xupd_body,
        grid=(n // _NBLK,),
        in_specs=[_rowspec(_NBLK, XP)]
        + [_rowspec(_NBLK, f.shape[1]) for f in flat],
        out_specs=_rowspec(_NBLK, XP),
        out_shape=jax.ShapeDtypeStruct((n, XP), jnp.float32),
        interpret=_INTERP,
    )(x, *flat)


def _readout_body(h_ref, w1, b1, w2, b2, w3, b3, out):
    z = _dot(h_ref[...], w1[...]) + b1[...]
    z = _silu_ln(z)
    z = _dot(z, w2[...]) + b2[...]
    z = _silu_ln(z)
    out[...] = _dot(z, w3[...]) + b3[...]


def _readout_call(h, ps):
    n = h.shape[0]
    d = ps[2]['w'].shape[1]
    args = [h, ps[0]['w'], ps[0]['b'].reshape(1, -1),
            ps[1]['w'], ps[1]['b'].reshape(1, -1),
            ps[2]['w'], ps[2]['b'].reshape(1, -1)]
    return pl.pallas_call(
        _readout_body,
        grid=(n // _NBLK,),
        in_specs=[_rowspec(_NBLK, H)] + [_fs(a.shape) for a in args[1:]],
        out_specs=_rowspec(_NBLK, d),
        out_shape=jax.ShapeDtypeStruct((n, d), jnp.float32),
        interpret=_INTERP,
    )(*args)


_VMESH = plsc.VectorSubcoreMesh(core_axis_name="c", subcore_axis_name="s")
_W = 128
_NSUB = 16
_NCORE = 2


def _row_share(n, sid):
    per = ((n + _NSUB - 1) // _NSUB + 7) // 8 * 8
    last = n - per * (_NSUB - 1)
    assert last > 0 and last % 8 == 0
    return per, last


def _stage_rows(src, dst, n, sid):
    per, last = _row_share(n, sid)

    @pl.when(sid < _NSUB - 1)
    def _():
        sl = pl.ds(sid * per, per)
        pltpu.sync_copy(src.at[sl], dst.at[sl])

    @pl.when(sid == _NSUB - 1)
    def _():
        sl = pl.ds((_NSUB - 1) * per, last)
        pltpu.sync_copy(src.at[sl], dst.at[sl])


def _gather_rows(table, idx):
    n, D = table.shape
    E = idx.shape[0]
    nwin = E // _W
    nwin_c = nwin // _NCORE
    idx2 = idx.reshape(nwin, _W)

    @functools.partial(
        pl.kernel,
        out_type=jax.ShapeDtypeStruct((E, D), jnp.float32),
        mesh=_VMESH,
        compiler_params=pltpu.CompilerParams(use_tc_tiling_on_sc=False),
        scratch_types=[pltpu.VMEM((_W,), jnp.int32),
                       pltpu.VMEM((_W, D), jnp.float32),
                       pltpu.VMEM_SHARED((n, D), jnp.float32)],
        name=f"sc_gather_{E}_{D}",
    )
    def k(table_hbm, idx_hbm, out_hbm, i_v, rows_v, tab_sh):
        cid = lax.axis_index("c")
        sid = lax.axis_index("s")

        _stage_rows(table_hbm, tab_sh, n, sid)
        plsc.subcore_barrier()

        @pl.loop(sid, nwin_c, step=_NSUB)
        def _(w):
            g = cid * nwin_c + w
            pltpu.sync_copy(idx_hbm.at[g], i_v)
            pltpu.sync_copy(tab_sh.at[i_v], rows_v)
            pltpu.sync_copy(rows_v, out_hbm.at[pl.ds(g * _W, _W)])

    return k(table, idx2)


def _scatter_add(vals, idx, n):
    E, D = vals.shape
    nwin = E // _W
    nwin_c = nwin // _NCORE
    idx2 = idx.reshape(nwin, _W)
    zeros = jnp.zeros((n, D), jnp.float32)

    @functools.partial(
        pl.kernel,
        out_type=jax.ShapeDtypeStruct((_NCORE, n, D), jnp.float32),
        mesh=_VMESH,
        compiler_params=pltpu.CompilerParams(use_tc_tiling_on_sc=False),
        scratch_types=[pltpu.VMEM((_W,), jnp.int32),
                       pltpu.VMEM((_W, D), jnp.float32),
                       pltpu.VMEM_SHARED((n, D), jnp.float32)],
        name=f"sc_scatter_{E}_{D}",
    )
    def k(vals_hbm, idx_hbm, zeros_hbm, out_hbm, i_v, v_v, acc_sh):
        cid = lax.axis_index("c")
        sid = lax.axis_index("s")

        _stage_rows(zeros_hbm, acc_sh, n, sid)
        plsc.subcore_barrier()

        @pl.loop(sid, nwin_c, step=_NSUB)
        def _(w):
            g = cid * nwin_c + w
            pltpu.sync_copy(idx_hbm.at[g], i_v)
            pltpu.sync_copy(vals_hbm.at[pl.ds(g * _W, _W)], v_v)
            pltpu.sync_copy(v_v, acc_sh.at[i_v], add=True)

        plsc.subcore_barrier()
        _stage_rows(acc_sh, out_hbm.at[cid], n, sid)

    return k(vals, idx2, zeros)


def kernel(rec_h, rec_x, rec_e_index, rec_e_type, rec_batch, lig_h_type,
           lig_x, lig_e_index, lig_e_type, lig_batch, timestep,
           inter_e_index, inter_e_type, params):
    t = jnp.asarray(_PE)[timestep]
    for i, p in enumerate(params['embd_timestep']):
        t = t @ p['w'] + p['b']
        t = jax.nn.silu(_ln(t))

    rsf = jnp.concatenate([rec_e_index[0], rec_e_index[1]])
    rdf = jnp.concatenate([rec_e_index[1], rec_e_index[0]])
    lsf = jnp.concatenate([lig_e_index[0], lig_e_index[1]])
    ldf = jnp.concatenate([lig_e_index[1], lig_e_index[0]])
    isrc, idst = inter_e_index[0], inter_e_index[1]

    rec_xp = jnp.pad(rec_x, ((0, 0), (0, XP - 3)))
    lig_xp = jnp.pad(lig_x, ((0, 0), (0, XP - 3)))

    rec_h_cur = _rec_init_call(rec_h, params['embd_rec_h'])
    lig_h, tB = _lig_init_call(lig_h_type, lig_batch, t,
                               params['embd_lig_h'], params['merge_lig_h'])

    full_rec_e = _onehot_embed(jnp.concatenate([rec_e_type, rec_e_type]),
                               params['embd_rec_e'])
    lig_e_raw = _onehot_embed(jnp.concatenate([lig_e_type, lig_e_type]),
                              params['embd_lig_e'])
    inter_e_raw = _onehot_embed(inter_e_type, params['embd_inter_e'])
    t_lig_e = _gather_rows(tB, ldf)
    t_inter_e = _gather_rows(tB, idst)
    full_lig_e = _merge_e_call(lig_e_raw, t_lig_e, params['merge_lig_e'])
    inter_e = _merge_e_call(inter_e_raw, t_inter_e, params['merge_inter_e'])

    for lp in params['layers']:
        tab_r = jnp.concatenate([rec_h_cur, rec_xp], axis=1)
        gs = _gather_rows(tab_r, rsf)
        gd = _gather_rows(tab_r, rdf)
        m_rr, full_rec_e = _msg_call(gs, gd, full_rec_e,
                                     lp['rr_msg'], lp['rr_e'])
        acc_rr = _scatter_add(m_rr, rdf, N_REC)
        rec_h_cur = _upd_call(rec_h_cur, [acc_rr], lp['rr_upd'])

        tab_l = jnp.concatenate([lig_h, lig_xp], axis=1)
        gs = _gather_rows(tab_l, lsf)
        gd = _gather_rows(tab_l, ldf)
        msc_ll, full_lig_e = _msg_call(gs, gd, full_lig_e,
                                       lp['ll_msg'], lp['ll_e'], lp['ll_x'])
        acc_ll = _scatter_add(msc_ll, ldf, N_LIG)
        lig_xp = _xupd_call(lig_xp, acc_ll)

        tab_r2 = jnp.concatenate([rec_h_cur, rec_xp], axis=1)
        tab_l2 = jnp.concatenate([lig_h, lig_xp], axis=1)
        gr = _gather_rows(tab_r2, isrc)
        gl = _gather_rows(tab_l2, idst)
        msc_rl, inter_e = _msg_call(gr, gl, inter_e,
                                    lp['rl_msg'], lp['rl_e'], lp['rl_x'])
        acc_rl = _scatter_add(msc_rl, idst, N_LIG)
        lig_xp = _xupd_call(lig_xp, acc_rl)
        lig_h = _upd_call(lig_h, [acc_ll, acc_rl], lp['ll_upd'])

    lig_h_out = _readout_call(lig_h, params['readout_lig_h'])
    return lig_h_out, lig_xp[:, :3]


# speedup vs baseline: 2.9566x; 1.1964x over previous
import functools

import jax
import jax.numpy as jnp
import numpy as np
from jax import lax
from jax.experimental import pallas as pl
from jax.experimental.pallas import tpu as pltpu
from jax.experimental.pallas import tpu_sc as plsc

N_REC = 10000
N_LIG = 10000
H = 128
EE = 64
HID = 128
TD = 128
NT = 1000
XP = 16

_EBLK = 1000
_NBLK = 1000

_INTERP = False


def _pe_table_np(d, n):
    pos = np.arange(n)[:, None].astype(np.float64)
    i = np.arange(d)[None, :]
    ang = pos / np.power(10000.0, (2 * (i // 2)) / d)
    t = np.zeros((n, d))
    t[:, 0::2] = np.sin(ang[:, 0::2])
    t[:, 1::2] = np.cos(ang[:, 1::2])
    return t.astype(np.float32)


_PE = _pe_table_np(TD, NT)


def _ln(x):
    m = jnp.mean(x, axis=-1, keepdims=True)
    v = jnp.mean((x - m) * (x - m), axis=-1, keepdims=True)
    return (x - m) * lax.rsqrt(v + 1e-5)


def _silu_ln(x):
    y = _ln(x)
    return y * jax.nn.sigmoid(y)


def _fs(shape):
    nd = len(shape)
    return pl.BlockSpec(shape, lambda i, _n=nd: (0,) * _n)


def _rowspec(blk, width):
    return pl.BlockSpec((blk, width), lambda i: (i, 0))


def _dot(a, b):
    return jnp.dot(a, b, preferred_element_type=jnp.float32)


def _msg_body(has_x, gs_ref, gd_ref, e_ref,
              w1hs, w1hd, w1e, w1d, b1, w2, b2, we, be, *rest):
    if has_x:
        wx, bx, msc_out, e_out = rest
    else:
        msc_out, e_out = rest
    gs = gs_ref[...]
    gd = gd_ref[...]
    hs = gs[:, :H]
    hd = gd[:, :H]
    e = e_ref[...]
    diff = gd[:, H:] - gs[:, H:]
    d2 = jnp.sum(diff * diff, axis=1, keepdims=True)
    z = (_dot(hs, w1hs[...]) + _dot(hd, w1hd[...]) + _dot(e, w1e[...])
         + d2 * w1d[...] + b1[...])
    z = _silu_ln(z)
    z = _dot(z, w2[...]) + b2[...]
    m = _silu_ln(z)
    e_out[...] = e + _dot(m, we[...]) + be[...]
    if has_x:
        coef = jnp.sum(m * wx[...], axis=1, keepdims=True) + bx[...]
        sc = diff * coef
        col = lax.broadcasted_iota(jnp.int32, sc.shape, 1)
        sc = jnp.where(col == 3, 1.0, sc)
        msc_out[...] = jnp.concatenate([m, sc], axis=1)
    else:
        msc_out[...] = m


def _msg_call(gs, gd, e, msg_ps, e_ps, x_ps=None):
    E = gs.shape[0]
    has_x = x_ps is not None
    w1 = msg_ps[0]['w']
    w1hs = w1[:H]
    w1hd = w1[H:2 * H]
    w1e = w1[2 * H:2 * H + EE]
    w1d = w1[2 * H + EE:].reshape(1, HID)
    b1 = msg_ps[0]['b'].reshape(1, HID)
    w2 = msg_ps[1]['w']
    b2 = msg_ps[1]['b'].reshape(1, HID)
    we = e_ps['w']
    be = e_ps['b'].reshape(1, EE)
    args = [gs, gd, e, w1hs, w1hd, w1e, w1d, b1, w2, b2, we, be]
    mw = HID + XP if has_x else HID
    outs = [jax.ShapeDtypeStruct((E, mw), jnp.float32),
            jax.ShapeDtypeStruct((E, EE), jnp.float32)]
    out_specs = [_rowspec(_EBLK, mw), _rowspec(_EBLK, EE)]
    if has_x:
        args += [x_ps['w'].reshape(1, HID), x_ps['b'].reshape(1, 1)]
    in_specs = [_rowspec(_EBLK, H + XP), _rowspec(_EBLK, H + XP),
                _rowspec(_EBLK, EE)]
    in_specs += [_fs(a.shape) for a in args[3:]]
    return pl.pallas_call(
        functools.partial(_msg_body, has_x),
        grid=(E // _EBLK,),
        in_specs=in_specs,
        out_specs=out_specs,
        out_shape=outs,
        interpret=_INTERP,
    )(*args)


def _onehot_body(nvoc, t_ref, emb, out):
    oh = (lax.broadcasted_iota(jnp.int32, (t_ref.shape[0], nvoc), 1)
          == t_ref[...]).astype(jnp.float32)
    out[...] = _dot(oh, emb[...])


def _onehot_embed(types, emb):
    E = types.shape[0]
    nvoc = emb.shape[0]
    if nvoc % 8 != 0:
        emb = jnp.pad(emb, ((0, 8 - nvoc % 8), (0, 0)))
        nvoc = emb.shape[0]
    d = emb.shape[1]
    return pl.pallas_call(
        functools.partial(_onehot_body, nvoc),
        grid=(E // _EBLK,),
        in_specs=[_rowspec(_EBLK, 1), _fs(emb.shape)],
        out_specs=_rowspec(_EBLK, d),
        out_shape=jax.ShapeDtypeStruct((E, d), jnp.float32),
        interpret=_INTERP,
    )(types.reshape(E, 1), emb)


def _merge_e_body(e_ref, t_ref, wa, wb, b1, w2, b2, out):
    z = _dot(e_ref[...], wa[...]) + _dot(t_ref[...], wb[...]) + b1[...]
    z = _silu_ln(z)
    z = _dot(z, w2[...]) + b2[...]
    out[...] = _silu_ln(z)


def _merge_e_call(e, t, ps):
    E = e.shape[0]
    w1 = ps[0]['w']
    args = [e, t, w1[:EE], w1[EE:], ps[0]['b'].reshape(1, EE),
            ps[1]['w'], ps[1]['b'].reshape(1, EE)]
    return pl.pallas_call(
        _merge_e_body,
        grid=(E // _EBLK,),
        in_specs=[_rowspec(_EBLK, EE), _rowspec(_EBLK, TD)]
        + [_fs(a.shape) for a in args[2:]],
        out_specs=_rowspec(_EBLK, EE),
        out_shape=jax.ShapeDtypeStruct((E, EE), jnp.float32),
        interpret=_INTERP,
    )(*args)


def _rec_init_body(h_ref, w_ref, out):
    out[...] = _dot(h_ref[...], w_ref[...])


def _rec_init_call(rec_h, w):
    return pl.pallas_call(
        _rec_init_body,
        grid=(N_REC // _NBLK,),
        in_specs=[_rowspec(_NBLK, rec_h.shape[1]), _fs(w.shape)],
        out_specs=_rowspec(_NBLK, H),
        out_shape=jax.ShapeDtypeStruct((N_REC, H), jnp.float32),
        interpret=_INTERP,
    )(rec_h, w)


def _lig_init_body(ty_ref, ba_ref, t_ref, emb, wa, wb, b1, w2, b2,
                   h_out, tb_out):
    blk = ty_ref.shape[0]
    oh = (lax.broadcasted_iota(jnp.int32, (blk, 32), 1)
          == ty_ref[...]).astype(jnp.float32)
    h0 = _dot(oh, emb[...])
    ohb = (lax.broadcasted_iota(jnp.int32, (blk, 8), 1)
           == ba_ref[...]).astype(jnp.float32)
    tb = _dot(ohb, t_ref[...])
    tb_out[...] = tb
    z = _dot(h0, wa[...]) + _dot(tb, wb[...]) + b1[...]
    z = _silu_ln(z)
    z = _dot(z, w2[...]) + b2[...]
    h_out[...] = _silu_ln(z)


def _lig_init_call(lig_h_type, lig_batch, t, emb, ps):
    w1 = ps[0]['w']
    args = [lig_h_type.reshape(N_LIG, 1), lig_batch.reshape(N_LIG, 1), t,
            emb, w1[:H], w1[H:], ps[0]['b'].reshape(1, H),
            ps[1]['w'], ps[1]['b'].reshape(1, H)]
    return pl.pallas_call(
        _lig_init_body,
        grid=(N_LIG // _NBLK,),
        in_specs=[_rowspec(_NBLK, 1), _rowspec(_NBLK, 1)]
        + [_fs(a.shape) for a in args[2:]],
        out_specs=[_rowspec(_NBLK, H), _rowspec(_NBLK, TD)],
        out_shape=[jax.ShapeDtypeStruct((N_LIG, H), jnp.float32),
                   jax.ShapeDtypeStruct((N_LIG, TD), jnp.float32)],
        interpret=_INTERP,
    )(*args)


def _upd_body(nacc, h_ref, wa, wb, b1, w2, b2, *rest):
    accs, out = rest[:nacc], rest[nacc]
    h = h_ref[...]
    agg = accs[0][...][:, :H]
    for a in accs[1:]:
        agg = agg + a[...][:, :H]
    z = _dot(h, wa[...]) + _dot(agg, wb[...]) + b1[...]
    z = _silu_ln(z)
    z = _dot(z, w2[...]) + b2[...]
    out[...] = h + z


def _upd_call(h, accs, ps):
    n = h.shape[0]
    w1 = ps[0]['w']
    flat = []
    for acc in accs:
        for p in range(acc.shape[0]):
            flat.append(acc[p])
    args = [h, w1[:H], w1[H:], ps[0]['b'].reshape(1, HID),
            ps[1]['w'], ps[1]['b'].reshape(1, H)] + flat
    return pl.pallas_call(
        functools.partial(_upd_body, len(flat)),
        grid=(n // _NBLK,),
        in_specs=[_rowspec(_NBLK, H)] + [_fs(a.shape) for a in args[1:6]]
        + [_rowspec(_NBLK, f.shape[1]) for f in flat],
        out_specs=_rowspec(_NBLK, H),
        out_shape=jax.ShapeDtypeStruct((n, H), jnp.float32),
        interpret=_INTERP,
    )(*args)


def _xupd_body(x_ref, *rest):
    accs, out = rest[:-1], rest[-1]
    acc = accs[0][...]
    for a in accs[1:]:
        acc = acc + a[...]
    num = acc[:, HID:HID + XP]
    cnt = acc[:, HID + 3:HID + 4]
    col = lax.broadcasted_iota(jnp.int32, num.shape, 1)
    num = jnp.where(col == 3, 0.0, num)
    out[...] = x_ref[...] + num / (cnt + 1e-8)


def _xupd_call(x, acc):
    n = x.shape[0]
    flat = [acc[p] for p in range(acc.shape[0])]
    return pl.pallas_call(
        _xupd_body,
        grid=(n // _NBLK,),
        in_specs=[_rowspec(_NBLK, XP)]
        + [_rowspec(_NBLK, f.shape[1]) for f in flat],
        out_specs=_rowspec(_NBLK, XP),
        out_shape=jax.ShapeDtypeStruct((n, XP), jnp.float32),
        interpret=_INTERP,
    )(x, *flat)


def _readout_body(h_ref, w1, b1, w2, b2, w3, b3, out):
    z = _dot(h_ref[...], w1[...]) + b1[...]
    z = _silu_ln(z)
    z = _dot(z, w2[...]) + b2[...]
    z = _silu_ln(z)
    out[...] = _dot(z, w3[...]) + b3[...]


def _readout_call(h, ps):
    n = h.shape[0]
    d = ps[2]['w'].shape[1]
    args = [h, ps[0]['w'], ps[0]['b'].reshape(1, -1),
            ps[1]['w'], ps[1]['b'].reshape(1, -1),
            ps[2]['w'], ps[2]['b'].reshape(1, -1)]
    return pl.pallas_call(
        _readout_body,
        grid=(n // _NBLK,),
        in_specs=[_rowspec(_NBLK, H)] + [_fs(a.shape) for a in args[1:]],
        out_specs=_rowspec(_NBLK, d),
        out_shape=jax.ShapeDtypeStruct((n, d), jnp.float32),
        interpret=_INTERP,
    )(*args)


_VMESH = plsc.VectorSubcoreMesh(core_axis_name="c", subcore_axis_name="s")
_W = 128
_NSUB = 16
_NCORE = 2


def _row_share(n, sid):
    per = ((n + _NSUB - 1) // _NSUB + 7) // 8 * 8
    last = n - per * (_NSUB - 1)
    assert last > 0 and last % 8 == 0
    return per, last


def _stage_rows(src, dst, n, sid):
    per, last = _row_share(n, sid)

    @pl.when(sid < _NSUB - 1)
    def _():
        sl = pl.ds(sid * per, per)
        pltpu.sync_copy(src.at[sl], dst.at[sl])

    @pl.when(sid == _NSUB - 1)
    def _():
        sl = pl.ds((_NSUB - 1) * per, last)
        pltpu.sync_copy(src.at[sl], dst.at[sl])


def _gather_rows(table, idx):
    n, D = table.shape
    E = idx.shape[0]
    nwin = E // _W
    nwin_c = nwin // _NCORE
    idx2 = idx.reshape(nwin, _W)

    @functools.partial(
        pl.kernel,
        out_type=jax.ShapeDtypeStruct((E, D), jnp.float32),
        mesh=_VMESH,
        compiler_params=pltpu.CompilerParams(use_tc_tiling_on_sc=False),
        scratch_types=[pltpu.VMEM((_W,), jnp.int32),
                       pltpu.VMEM((_W, D), jnp.float32),
                       pltpu.VMEM_SHARED((n, D), jnp.float32)],
        name=f"sc_gather_{E}_{D}",
    )
    def k(table_hbm, idx_hbm, out_hbm, i_v, rows_v, tab_sh):
        cid = lax.axis_index("c")
        sid = lax.axis_index("s")

        _stage_rows(table_hbm, tab_sh, n, sid)
        plsc.subcore_barrier()

        @pl.loop(sid, nwin_c, step=_NSUB)
        def _(w):
            g = cid * nwin_c + w
            pltpu.sync_copy(idx_hbm.at[g], i_v)
            pltpu.sync_copy(tab_sh.at[i_v], rows_v)
            pltpu.sync_copy(rows_v, out_hbm.at[pl.ds(g * _W, _W)])

    return k(table, idx2)


def _scatter_add(pairs, n):
    D = pairs[0][0].shape[1]
    nwins = [v.shape[0] // _W for v, _ in pairs]
    flat = []
    for v, idx in pairs:
        flat += [v, idx.reshape(-1, _W)]
    zeros = jnp.zeros((n, D), jnp.float32)
    tag = "_".join(str(v.shape[0]) for v, _ in pairs)

    @functools.partial(
        pl.kernel,
        out_type=jax.ShapeDtypeStruct((_NCORE, n, D), jnp.float32),
        mesh=_VMESH,
        compiler_params=pltpu.CompilerParams(use_tc_tiling_on_sc=False),
        scratch_types=[pltpu.VMEM((_W,), jnp.int32),
                       pltpu.VMEM((_W, D), jnp.float32),
                       pltpu.VMEM_SHARED((n, D), jnp.float32)],
        name=f"sc_scatter_{tag}_{D}",
    )
    def k(*refs):
        *ins, out_hbm, i_v, v_v, acc_sh = refs
        zeros_hbm = ins[-1]
        cid = lax.axis_index("c")
        sid = lax.axis_index("s")

        _stage_rows(zeros_hbm, acc_sh, n, sid)
        plsc.subcore_barrier()

        for p, nwin in enumerate(nwins):
            vals_hbm = ins[2 * p]
            idx_hbm = ins[2 * p + 1]
            nwin_c = nwin // _NCORE

            @pl.loop(sid, nwin_c, step=_NSUB)
            def _(w, vals_hbm=vals_hbm, idx_hbm=idx_hbm, nwin_c=nwin_c):
                g = cid * nwin_c + w
                pltpu.sync_copy(idx_hbm.at[g], i_v)
                pltpu.sync_copy(vals_hbm.at[pl.ds(g * _W, _W)], v_v)
                pltpu.sync_copy(v_v, acc_sh.at[i_v], add=True)

        plsc.subcore_barrier()
        _stage_rows(acc_sh, out_hbm.at[cid], n, sid)

    return k(*flat, zeros)


def kernel(rec_h, rec_x, rec_e_index, rec_e_type, rec_batch, lig_h_type,
           lig_x, lig_e_index, lig_e_type, lig_batch, timestep,
           inter_e_index, inter_e_type, params):
    t = jnp.asarray(_PE)[timestep]
    for i, p in enumerate(params['embd_timestep']):
        t = t @ p['w'] + p['b']
        t = jax.nn.silu(_ln(t))

    r0, r1 = rec_e_index[0], rec_e_index[1]
    l0, l1 = lig_e_index[0], lig_e_index[1]
    isrc, idst = inter_e_index[0], inter_e_index[1]

    rec_xp = jnp.pad(rec_x, ((0, 0), (0, XP - 3)))
    lig_xp = jnp.pad(lig_x, ((0, 0), (0, XP - 3)))

    rec_h_cur = _rec_init_call(rec_h, params['embd_rec_h'])
    lig_h, tB = _lig_init_call(lig_h_type, lig_batch, t,
                               params['embd_lig_h'], params['merge_lig_h'])

    rec_e_raw = _onehot_embed(rec_e_type, params['embd_rec_e'])
    re = [rec_e_raw, rec_e_raw]
    lig_e_raw = _onehot_embed(lig_e_type, params['embd_lig_e'])
    inter_e_raw = _onehot_embed(inter_e_type, params['embd_inter_e'])
    tb0 = _gather_rows(tB, l0)
    tb1 = _gather_rows(tB, l1)
    le = [_merge_e_call(lig_e_raw, tb1, params['merge_lig_e']),
          _merge_e_call(lig_e_raw, tb0, params['merge_lig_e'])]
    t_inter_e = _gather_rows(tB, idst)
    inter_e = _merge_e_call(inter_e_raw, t_inter_e, params['merge_inter_e'])

    for lp in params['layers']:
        tab_r = jnp.concatenate([rec_h_cur, rec_xp], axis=1)
        g0 = _gather_rows(tab_r, r0)
        g1 = _gather_rows(tab_r, r1)
        m0, re0 = _msg_call(g0, g1, re[0], lp['rr_msg'], lp['rr_e'])
        m1, re1 = _msg_call(g1, g0, re[1], lp['rr_msg'], lp['rr_e'])
        re = [re0, re1]
        acc_rr = _scatter_add([(m0, r1), (m1, r0)], N_REC)
        rec_h_cur = _upd_call(rec_h_cur, [acc_rr], lp['rr_upd'])

        tab_l = jnp.concatenate([lig_h, lig_xp], axis=1)
        g0 = _gather_rows(tab_l, l0)
        g1 = _gather_rows(tab_l, l1)
        msc0, le0 = _msg_call(g0, g1, le[0], lp['ll_msg'], lp['ll_e'],
                              lp['ll_x'])
        msc1, le1 = _msg_call(g1, g0, le[1], lp['ll_msg'], lp['ll_e'],
                              lp['ll_x'])
        le = [le0, le1]
        acc_ll = _scatter_add([(msc0, l1), (msc1, l0)], N_LIG)
        lig_xp = _xupd_call(lig_xp, acc_ll)

        tab_r2 = jnp.concatenate([rec_h_cur, rec_xp], axis=1)
        tab_l2 = jnp.concatenate([lig_h, lig_xp], axis=1)
        gr = _gather_rows(tab_r2, isrc)
        gl = _gather_rows(tab_l2, idst)
        msc_rl, inter_e = _msg_call(gr, gl, inter_e,
                                    lp['rl_msg'], lp['rl_e'], lp['rl_x'])
        acc_rl = _scatter_add([(msc_rl, idst)], N_LIG)
        lig_xp = _xupd_call(lig_xp, acc_rl)
        lig_h = _upd_call(lig_h, [acc_ll, acc_rl], lp['ll_upd'])

    lig_h_out = _readout_call(lig_h, params['readout_lig_h'])
    return lig_h_out, lig_xp[:, :3]


# speedup vs baseline: 3.0581x; 1.0343x over previous
import functools

import jax
import jax.numpy as jnp
import numpy as np
from jax import lax
from jax.experimental import pallas as pl
from jax.experimental.pallas import tpu as pltpu
from jax.experimental.pallas import tpu_sc as plsc

N_REC = 10000
N_LIG = 10000
H = 128
EE = 64
HID = 128
TD = 128
NT = 1000
XP = 16

_EBLK = 1000
_NBLK = 1000

_INTERP = False


def _pe_table_np(d, n):
    pos = np.arange(n)[:, None].astype(np.float64)
    i = np.arange(d)[None, :]
    ang = pos / np.power(10000.0, (2 * (i // 2)) / d)
    t = np.zeros((n, d))
    t[:, 0::2] = np.sin(ang[:, 0::2])
    t[:, 1::2] = np.cos(ang[:, 1::2])
    return t.astype(np.float32)


_PE = _pe_table_np(TD, NT)


def _ln(x):
    m = jnp.mean(x, axis=-1, keepdims=True)
    v = jnp.mean((x - m) * (x - m), axis=-1, keepdims=True)
    return (x - m) * lax.rsqrt(v + 1e-5)


def _silu_ln(x):
    y = _ln(x)
    return y * jax.nn.sigmoid(y)


def _fs(shape):
    nd = len(shape)
    return pl.BlockSpec(shape, lambda i, _n=nd: (0,) * _n)


def _rowspec(blk, width):
    return pl.BlockSpec((blk, width), lambda i: (i, 0))


def _dot(a, b):
    return jnp.dot(a, b, preferred_element_type=jnp.float32)


def _msg_body(has_x, gs_ref, gd_ref, e_ref,
              w1hs, w1hd, w1e, w1d, b1, w2, b2, we, be, *rest):
    if has_x:
        wx, bx, msc_out, e_out = rest
    else:
        msc_out, e_out = rest
    gs = gs_ref[...]
    gd = gd_ref[...]
    hs = gs[:, :H]
    hd = gd[:, :H]
    e = e_ref[...]
    diff = gd[:, H:] - gs[:, H:]
    d2 = jnp.sum(diff * diff, axis=1, keepdims=True)
    z = (_dot(hs, w1hs[...]) + _dot(hd, w1hd[...]) + _dot(e, w1e[...])
         + d2 * w1d[...] + b1[...])
    z = _silu_ln(z)
    z = _dot(z, w2[...]) + b2[...]
    m = _silu_ln(z)
    e_out[...] = e + _dot(m, we[...]) + be[...]
    if has_x:
        coef = jnp.sum(m * wx[...], axis=1, keepdims=True) + bx[...]
        sc = diff * coef
        col = lax.broadcasted_iota(jnp.int32, sc.shape, 1)
        sc = jnp.where(col == 3, 1.0, sc)
        msc_out[...] = jnp.concatenate([m, sc], axis=1)
    else:
        msc_out[...] = m


def _msg_call(gs, gd, e, msg_ps, e_ps, x_ps=None):
    E = gs.shape[0]
    has_x = x_ps is not None
    w1 = msg_ps[0]['w']
    w1hs = w1[:H]
    w1hd = w1[H:2 * H]
    w1e = w1[2 * H:2 * H + EE]
    w1d = w1[2 * H + EE:].reshape(1, HID)
    b1 = msg_ps[0]['b'].reshape(1, HID)
    w2 = msg_ps[1]['w']
    b2 = msg_ps[1]['b'].reshape(1, HID)
    we = e_ps['w']
    be = e_ps['b'].reshape(1, EE)
    args = [gs, gd, e, w1hs, w1hd, w1e, w1d, b1, w2, b2, we, be]
    mw = HID + XP if has_x else HID
    outs = [jax.ShapeDtypeStruct((E, mw), jnp.float32),
            jax.ShapeDtypeStruct((E, EE), jnp.float32)]
    out_specs = [_rowspec(_EBLK, mw), _rowspec(_EBLK, EE)]
    if has_x:
        args += [x_ps['w'].reshape(1, HID), x_ps['b'].reshape(1, 1)]
    in_specs = [_rowspec(_EBLK, H + XP), _rowspec(_EBLK, H + XP),
                _rowspec(_EBLK, EE)]
    in_specs += [_fs(a.shape) for a in args[3:]]
    return pl.pallas_call(
        functools.partial(_msg_body, has_x),
        grid=(E // _EBLK,),
        in_specs=in_specs,
        out_specs=out_specs,
        out_shape=outs,
        interpret=_INTERP,
    )(*args)


def _onehot_body(nvoc, t_ref, emb, out):
    oh = (lax.broadcasted_iota(jnp.int32, (t_ref.shape[0], nvoc), 1)
          == t_ref[...]).astype(jnp.float32)
    out[...] = _dot(oh, emb[...])


def _onehot_embed(types, emb):
    E = types.shape[0]
    nvoc = emb.shape[0]
    if nvoc % 8 != 0:
        emb = jnp.pad(emb, ((0, 8 - nvoc % 8), (0, 0)))
        nvoc = emb.shape[0]
    d = emb.shape[1]
    return pl.pallas_call(
        functools.partial(_onehot_body, nvoc),
        grid=(E // _EBLK,),
        in_specs=[_rowspec(_EBLK, 1), _fs(emb.shape)],
        out_specs=_rowspec(_EBLK, d),
        out_shape=jax.ShapeDtypeStruct((E, d), jnp.float32),
        interpret=_INTERP,
    )(types.reshape(E, 1), emb)


def _merge_e_body(e_ref, t_ref, wa, wb, b1, w2, b2, out):
    z = _dot(e_ref[...], wa[...]) + _dot(t_ref[...], wb[...]) + b1[...]
    z = _silu_ln(z)
    z = _dot(z, w2[...]) + b2[...]
    out[...] = _silu_ln(z)


def _merge_e_call(e, t, ps):
    E = e.shape[0]
    w1 = ps[0]['w']
    args = [e, t, w1[:EE], w1[EE:], ps[0]['b'].reshape(1, EE),
            ps[1]['w'], ps[1]['b'].reshape(1, EE)]
    return pl.pallas_call(
        _merge_e_body,
        grid=(E // _EBLK,),
        in_specs=[_rowspec(_EBLK, EE), _rowspec(_EBLK, TD)]
        + [_fs(a.shape) for a in args[2:]],
        out_specs=_rowspec(_EBLK, EE),
        out_shape=jax.ShapeDtypeStruct((E, EE), jnp.float32),
        interpret=_INTERP,
    )(*args)


def _rec_init_body(h_ref, w_ref, out):
    out[...] = _dot(h_ref[...], w_ref[...])


def _rec_init_call(rec_h, w):
    return pl.pallas_call(
        _rec_init_body,
        grid=(N_REC // _NBLK,),
        in_specs=[_rowspec(_NBLK, rec_h.shape[1]), _fs(w.shape)],
        out_specs=_rowspec(_NBLK, H),
        out_shape=jax.ShapeDtypeStruct((N_REC, H), jnp.float32),
        interpret=_INTERP,
    )(rec_h, w)


def _lig_init_body(ty_ref, ba_ref, t_ref, emb, wa, wb, b1, w2, b2,
                   h_out, tb_out):
    blk = ty_ref.shape[0]
    oh = (lax.broadcasted_iota(jnp.int32, (blk, 32), 1)
          == ty_ref[...]).astype(jnp.float32)
    h0 = _dot(oh, emb[...])
    ohb = (lax.broadcasted_iota(jnp.int32, (blk, 8), 1)
           == ba_ref[...]).astype(jnp.float32)
    tb = _dot(ohb, t_ref[...])
    tb_out[...] = tb
    z = _dot(h0, wa[...]) + _dot(tb, wb[...]) + b1[...]
    z = _silu_ln(z)
    z = _dot(z, w2[...]) + b2[...]
    h_out[...] = _silu_ln(z)


def _lig_init_call(lig_h_type, lig_batch, t, emb, ps):
    w1 = ps[0]['w']
    args = [lig_h_type.reshape(N_LIG, 1), lig_batch.reshape(N_LIG, 1), t,
            emb, w1[:H], w1[H:], ps[0]['b'].reshape(1, H),
            ps[1]['w'], ps[1]['b'].reshape(1, H)]
    return pl.pallas_call(
        _lig_init_body,
        grid=(N_LIG // _NBLK,),
        in_specs=[_rowspec(_NBLK, 1), _rowspec(_NBLK, 1)]
        + [_fs(a.shape) for a in args[2:]],
        out_specs=[_rowspec(_NBLK, H), _rowspec(_NBLK, TD)],
        out_shape=[jax.ShapeDtypeStruct((N_LIG, H), jnp.float32),
                   jax.ShapeDtypeStruct((N_LIG, TD), jnp.float32)],
        interpret=_INTERP,
    )(*args)


def _upd_body(nacc, h_ref, wa, wb, b1, w2, b2, *rest):
    accs, out = rest[:nacc], rest[nacc]
    h = h_ref[...]
    agg = accs[0][...][:, :H]
    for a in accs[1:]:
        agg = agg + a[...][:, :H]
    z = _dot(h, wa[...]) + _dot(agg, wb[...]) + b1[...]
    z = _silu_ln(z)
    z = _dot(z, w2[...]) + b2[...]
    out[...] = h + z


def _upd_call(h, accs, ps):
    n = h.shape[0]
    w1 = ps[0]['w']
    flat = []
    for acc in accs:
        for p in range(acc.shape[0]):
            flat.append(acc[p])
    args = [h, w1[:H], w1[H:], ps[0]['b'].reshape(1, HID),
            ps[1]['w'], ps[1]['b'].reshape(1, H)] + flat
    return pl.pallas_call(
        functools.partial(_upd_body, len(flat)),
        grid=(n // _NBLK,),
        in_specs=[_rowspec(_NBLK, H)] + [_fs(a.shape) for a in args[1:6]]
        + [_rowspec(_NBLK, f.shape[1]) for f in flat],
        out_specs=_rowspec(_NBLK, H),
        out_shape=jax.ShapeDtypeStruct((n, H), jnp.float32),
        interpret=_INTERP,
    )(*args)


def _xupd_body(x_ref, *rest):
    accs, out = rest[:-1], rest[-1]
    acc = accs[0][...]
    for a in accs[1:]:
        acc = acc + a[...]
    num = acc[:, HID:HID + XP]
    cnt = acc[:, HID + 3:HID + 4]
    col = lax.broadcasted_iota(jnp.int32, num.shape, 1)
    num = jnp.where(col == 3, 0.0, num)
    out[...] = x_ref[...] + num / (cnt + 1e-8)


def _xupd_call(x, acc):
    n = x.shape[0]
    flat = [acc[p] for p in range(acc.shape[0])]
    return pl.pallas_call(
        _xupd_body,
        grid=(n // _NBLK,),
        in_specs=[_rowspec(_NBLK, XP)]
        + [_rowspec(_NBLK, f.shape[1]) for f in flat],
        out_specs=_rowspec(_NBLK, XP),
        out_shape=jax.ShapeDtypeStruct((n, XP), jnp.float32),
        interpret=_INTERP,
    )(x, *flat)


def _readout_body(h_ref, w1, b1, w2, b2, w3, b3, out):
    z = _dot(h_ref[...], w1[...]) + b1[...]
    z = _silu_ln(z)
    z = _dot(z, w2[...]) + b2[...]
    z = _silu_ln(z)
    out[...] = _dot(z, w3[...]) + b3[...]


def _readout_call(h, ps):
    n = h.shape[0]
    d = ps[2]['w'].shape[1]
    args = [h, ps[0]['w'], ps[0]['b'].reshape(1, -1),
            ps[1]['w'], ps[1]['b'].reshape(1, -1),
            ps[2]['w'], ps[2]['b'].reshape(1, -1)]
    return pl.pallas_call(
        _readout_body,
        grid=(n // _NBLK,),
        in_specs=[_rowspec(_NBLK, H)] + [_fs(a.shape) for a in args[1:]],
        out_specs=_rowspec(_NBLK, d),
        out_shape=jax.ShapeDtypeStruct((n, d), jnp.float32),
        interpret=_INTERP,
    )(*args)


_VMESH = plsc.VectorSubcoreMesh(core_axis_name="c", subcore_axis_name="s")
_W = 80
_NSUB = 16
_NCORE = 2


def _row_share(n, sid):
    per = ((n + _NSUB - 1) // _NSUB + 7) // 8 * 8
    last = n - per * (_NSUB - 1)
    assert last > 0 and last % 8 == 0
    return per, last


def _stage_rows(src, dst, n, sid):
    per, last = _row_share(n, sid)

    @pl.when(sid < _NSUB - 1)
    def _():
        sl = pl.ds(sid * per, per)
        pltpu.sync_copy(src.at[sl], dst.at[sl])

    @pl.when(sid == _NSUB - 1)
    def _():
        sl = pl.ds((_NSUB - 1) * per, last)
        pltpu.sync_copy(src.at[sl], dst.at[sl])


_NW = _NCORE * _NSUB


def _widx(idx):
    nwin = idx.shape[0] // _W
    nk = -(-nwin // _NW)
    idx2 = idx.reshape(nwin, _W)
    idx2 = jnp.pad(idx2, ((0, nk * _NW - nwin), (0, 0)))
    return idx2.reshape(nk, _NW, _W).transpose(1, 0, 2), nwin, nk


def _pipe2(n, start1, wait1, start2, wait2):
    if n > 0:
        start1(0, 0)
    if n > 1:
        start1(1, 1)

    def halfstep(k, b):
        wait1(k, b)
        start2(k, b)

        @pl.when(k + 2 < n)
        def _():
            wait2(k, b)
            start1(k + 2, b)

    @pl.loop(0, n // 2)
    def _(p):
        halfstep(2 * p, 0)
        halfstep(2 * p + 1, 1)

    if n % 2 == 1:
        if n >= 2:
            wait2(n - 2, 1)
        wait1(n - 1, 0)
        start2(n - 1, 0)
        wait2(n - 1, 0)
    else:
        if n >= 2:
            wait2(n - 2, 0)
        if n >= 1:
            wait2(n - 1, 1)


def _gather_rows(table, idx):
    n, D = table.shape
    E = idx.shape[0]
    idx3, nwin, nk = _widx(idx)
    n_main = nwin // _NW
    rem = nwin % _NW

    @functools.partial(
        pl.kernel,
        out_type=jax.ShapeDtypeStruct((E, D), jnp.float32),
        mesh=_VMESH,
        compiler_params=pltpu.CompilerParams(use_tc_tiling_on_sc=False),
        scratch_types=[pltpu.VMEM((nk, _W), jnp.int32),
                       pltpu.VMEM((_W, D), jnp.float32),
                       pltpu.VMEM((_W, D), jnp.float32),
                       pltpu.VMEM_SHARED((n, D), jnp.float32),
                       pltpu.SemaphoreType.DMA,
                       pltpu.SemaphoreType.DMA,
                       pltpu.SemaphoreType.DMA,
                       pltpu.SemaphoreType.DMA],
        name=f"sc_gather_{E}_{D}",
    )
    def k(table_hbm, idx_hbm, out_hbm, i_all, rows_a, rows_b, tab_sh,
          s1a, s1b, s2a, s2b):
        cid = lax.axis_index("c")
        sid = lax.axis_index("s")
        wid = sid * _NCORE + cid
        bufs = (rows_a, rows_b)
        s1 = (s1a, s1b)
        s2 = (s2a, s2b)

        pltpu.sync_copy(idx_hbm.at[wid], i_all)
        _stage_rows(table_hbm, tab_sh, n, sid)
        plsc.subcore_barrier()

        def start1(kk, b):
            pltpu.async_copy(tab_sh.at[i_all.at[kk]], bufs[b], s1[b])

        def wait1(kk, b):
            pltpu.make_async_copy(tab_sh.at[i_all.at[kk]], bufs[b],
                                  s1[b]).wait()

        def _out(kk):
            return out_hbm.at[pl.ds((kk * _NW + wid) * _W, _W)]

        def start2(kk, b):
            pltpu.async_copy(bufs[b], _out(kk), s2[b])

        def wait2(kk, b):
            pltpu.make_async_copy(bufs[b], _out(kk), s2[b]).wait()

        _pipe2(n_main, start1, wait1, start2, wait2)

        if rem:
            @pl.when(wid < rem)
            def _():
                pltpu.sync_copy(tab_sh.at[i_all.at[n_main]], rows_a)
                pltpu.sync_copy(rows_a, _out(n_main))

    return k(table, idx3)


def _scatter_add(pairs, n):
    D = pairs[0][0].shape[1]
    flat = []
    meta = []
    nk_max = 0
    for v, idx in pairs:
        idx3, nwin, nk = _widx(idx)
        flat += [v, idx3]
        meta.append((nwin // _NW, nwin % _NW, nk))
        nk_max = max(nk_max, nk)
    zeros = jnp.zeros((n, D), jnp.float32)
    tag = "_".join(str(v.shape[0]) for v, _ in pairs)

    @functools.partial(
        pl.kernel,
        out_type=jax.ShapeDtypeStruct((_NCORE, n, D), jnp.float32),
        mesh=_VMESH,
        compiler_params=pltpu.CompilerParams(use_tc_tiling_on_sc=False),
        scratch_types=[pltpu.VMEM((nk_max, _W), jnp.int32),
                       pltpu.VMEM((_W, D), jnp.float32),
                       pltpu.VMEM((_W, D), jnp.float32),
                       pltpu.VMEM_SHARED((n, D), jnp.float32),
                       pltpu.SemaphoreType.DMA,
                       pltpu.SemaphoreType.DMA,
                       pltpu.SemaphoreType.DMA,
                       pltpu.SemaphoreType.DMA],
        name=f"sc_scatter_{tag}_{D}",
    )
    def k(*refs):
        *ins, out_hbm, i_all, v_a, v_b, acc_sh, s1a, s1b, s2a, s2b = refs
        zeros_hbm = ins[-1]
        cid = lax.axis_index("c")
        sid = lax.axis_index("s")
        wid = sid * _NCORE + cid
        bufs = (v_a, v_b)
        s1 = (s1a, s1b)
        s2 = (s2a, s2b)

        _stage_rows(zeros_hbm, acc_sh, n, sid)
        plsc.subcore_barrier()

        for p, (n_main, rem, nk) in enumerate(meta):
            vals_hbm = ins[2 * p]
            idx_hbm = ins[2 * p + 1]
            pltpu.sync_copy(idx_hbm.at[wid], i_all.at[pl.ds(0, nk)])

            def _src(kk, vals_hbm=vals_hbm):
                return vals_hbm.at[pl.ds((kk * _NW + wid) * _W, _W)]

            def start1(kk, b):
                pltpu.async_copy(_src(kk), bufs[b], s1[b])

            def wait1(kk, b):
                pltpu.make_async_copy(_src(kk), bufs[b], s1[b]).wait()

            def start2(kk, b):
                pltpu.async_copy(bufs[b], acc_sh.at[i_all.at[kk]], s2[b],
                                 add=True)

            def wait2(kk, b):
                pltpu.make_async_copy(bufs[b], acc_sh.at[i_all.at[kk]],
                                      s2[b]).wait()

            _pipe2(n_main, start1, wait1, start2, wait2)

            if rem:
                @pl.when(wid < rem)
                def _():
                    pltpu.sync_copy(_src(n_main), v_a)
                    pltpu.sync_copy(v_a, acc_sh.at[i_all.at[n_main]],
                                    add=True)

        plsc.subcore_barrier()
        _stage_rows(acc_sh, out_hbm.at[cid], n, sid)

    return k(*flat, zeros)


def kernel(rec_h, rec_x, rec_e_index, rec_e_type, rec_batch, lig_h_type,
           lig_x, lig_e_index, lig_e_type, lig_batch, timestep,
           inter_e_index, inter_e_type, params):
    t = jnp.asarray(_PE)[timestep]
    for i, p in enumerate(params['embd_timestep']):
        t = t @ p['w'] + p['b']
        t = jax.nn.silu(_ln(t))

    r0, r1 = rec_e_index[0], rec_e_index[1]
    l0, l1 = lig_e_index[0], lig_e_index[1]
    isrc, idst = inter_e_index[0], inter_e_index[1]

    rec_xp = jnp.pad(rec_x, ((0, 0), (0, XP - 3)))
    lig_xp = jnp.pad(lig_x, ((0, 0), (0, XP - 3)))

    rec_h_cur = _rec_init_call(rec_h, params['embd_rec_h'])
    lig_h, tB = _lig_init_call(lig_h_type, lig_batch, t,
                               params['embd_lig_h'], params['merge_lig_h'])

    rec_e_raw = _onehot_embed(rec_e_type, params['embd_rec_e'])
    re = [rec_e_raw, rec_e_raw]
    lig_e_raw = _onehot_embed(lig_e_type, params['embd_lig_e'])
    inter_e_raw = _onehot_embed(inter_e_type, params['embd_inter_e'])
    tb0 = _gather_rows(tB, l0)
    tb1 = _gather_rows(tB, l1)
    le = [_merge_e_call(lig_e_raw, tb1, params['merge_lig_e']),
          _merge_e_call(lig_e_raw, tb0, params['merge_lig_e'])]
    t_inter_e = _gather_rows(tB, idst)
    inter_e = _merge_e_call(inter_e_raw, t_inter_e, params['merge_inter_e'])

    for lp in params['layers']:
        tab_r = jnp.concatenate([rec_h_cur, rec_xp], axis=1)
        g0 = _gather_rows(tab_r, r0)
        g1 = _gather_rows(tab_r, r1)
        m0, re0 = _msg_call(g0, g1, re[0], lp['rr_msg'], lp['rr_e'])
        m1, re1 = _msg_call(g1, g0, re[1], lp['rr_msg'], lp['rr_e'])
        re = [re0, re1]
        acc_rr = _scatter_add([(m0, r1), (m1, r0)], N_REC)
        rec_h_cur = _upd_call(rec_h_cur, [acc_rr], lp['rr_upd'])

        tab_l = jnp.concatenate([lig_h, lig_xp], axis=1)
        g0 = _gather_rows(tab_l, l0)
        g1 = _gather_rows(tab_l, l1)
        msc0, le0 = _msg_call(g0, g1, le[0], lp['ll_msg'], lp['ll_e'],
                              lp['ll_x'])
        msc1, le1 = _msg_call(g1, g0, le[1], lp['ll_msg'], lp['ll_e'],
                              lp['ll_x'])
        le = [le0, le1]
        acc_ll = _scatter_add([(msc0, l1), (msc1, l0)], N_LIG)
        lig_xp = _xupd_call(lig_xp, acc_ll)

        tab_r2 = jnp.concatenate([rec_h_cur, rec_xp], axis=1)
        tab_l2 = jnp.concatenate([lig_h, lig_xp], axis=1)
        gr = _gather_rows(tab_r2, isrc)
        gl = _gather_rows(tab_l2, idst)
        msc_rl, inter_e = _msg_call(gr, gl, inter_e,
                                    lp['rl_msg'], lp['rl_e'], lp['rl_x'])
        acc_rl = _scatter_add([(msc_rl, idst)], N_LIG)
        lig_xp = _xupd_call(lig_xp, acc_rl)
        lig_h = _upd_call(lig_h, [acc_ll, acc_rl], lp['ll_upd'])

    lig_h_out = _readout_call(lig_h, params['readout_lig_h'])
    return lig_h_out, lig_xp[:, :3]


# speedup vs baseline: 3.6732x; 1.2011x over previous
import functools

import jax
import jax.numpy as jnp
import numpy as np
from jax import lax
from jax.experimental import pallas as pl
from jax.experimental.pallas import tpu as pltpu
from jax.experimental.pallas import tpu_sc as plsc

N_REC = 10000
N_LIG = 10000
H = 128
EE = 64
HID = 128
TD = 128
NT = 1000
XP = 16

_EBLK = 2000
_NBLK = 1000

_INTERP = False


def _pe_table_np(d, n):
    pos = np.arange(n)[:, None].astype(np.float64)
    i = np.arange(d)[None, :]
    ang = pos / np.power(10000.0, (2 * (i // 2)) / d)
    t = np.zeros((n, d))
    t[:, 0::2] = np.sin(ang[:, 0::2])
    t[:, 1::2] = np.cos(ang[:, 1::2])
    return t.astype(np.float32)


_PE = _pe_table_np(TD, NT)


def _ln(x):
    m = jnp.mean(x, axis=-1, keepdims=True)
    v = jnp.mean((x - m) * (x - m), axis=-1, keepdims=True)
    return (x - m) * lax.rsqrt(v + 1e-5)


def _silu_ln(x):
    y = _ln(x)
    return y * jax.nn.sigmoid(y)


def _fs(shape):
    nd = len(shape)
    return pl.BlockSpec(shape, lambda i, _n=nd: (0,) * _n)


def _rowspec(blk, width):
    return pl.BlockSpec((blk, width), lambda i: (i, 0))


def _dot(a, b):
    return jnp.dot(a, b, preferred_element_type=jnp.float32)


def _msg_body(has_x, gs_ref, gd_ref, e_ref,
              w1hs, w1hd, w1e, w1d, b1, w2, b2, we, be, *rest):
    if has_x:
        wx, bx, msc_out, e_out = rest
    else:
        msc_out, e_out = rest
    gs = gs_ref[...]
    gd = gd_ref[...]
    hs = gs[:, :H]
    hd = gd[:, :H]
    e = e_ref[...]
    diff = gd[:, H:] - gs[:, H:]
    d2 = jnp.sum(diff * diff, axis=1, keepdims=True)
    z = (_dot(hs, w1hs[...]) + _dot(hd, w1hd[...]) + _dot(e, w1e[...])
         + d2 * w1d[...] + b1[...])
    z = _silu_ln(z)
    z = _dot(z, w2[...]) + b2[...]
    m = _silu_ln(z)
    e_out[...] = e + _dot(m, we[...]) + be[...]
    if has_x:
        coef = jnp.sum(m * wx[...], axis=1, keepdims=True) + bx[...]
        sc = diff * coef
        col = lax.broadcasted_iota(jnp.int32, sc.shape, 1)
        sc = jnp.where(col == 3, 1.0, sc)
        msc_out[...] = jnp.concatenate([m, sc], axis=1)
    else:
        msc_out[...] = m


def _msg_call(gs, gd, e, msg_ps, e_ps, x_ps=None):
    E = gs.shape[0]
    has_x = x_ps is not None
    w1 = msg_ps[0]['w']
    w1hs = w1[:H]
    w1hd = w1[H:2 * H]
    w1e = w1[2 * H:2 * H + EE]
    w1d = w1[2 * H + EE:].reshape(1, HID)
    b1 = msg_ps[0]['b'].reshape(1, HID)
    w2 = msg_ps[1]['w']
    b2 = msg_ps[1]['b'].reshape(1, HID)
    we = e_ps['w']
    be = e_ps['b'].reshape(1, EE)
    args = [gs, gd, e, w1hs, w1hd, w1e, w1d, b1, w2, b2, we, be]
    mw = HID + XP if has_x else HID
    outs = [jax.ShapeDtypeStruct((E, mw), jnp.float32),
            jax.ShapeDtypeStruct((E, EE), jnp.float32)]
    out_specs = [_rowspec(_EBLK, mw), _rowspec(_EBLK, EE)]
    if has_x:
        args += [x_ps['w'].reshape(1, HID), x_ps['b'].reshape(1, 1)]
    in_specs = [_rowspec(_EBLK, H + XP), _rowspec(_EBLK, H + XP),
                _rowspec(_EBLK, EE)]
    in_specs += [_fs(a.shape) for a in args[3:]]
    return pl.pallas_call(
        functools.partial(_msg_body, has_x),
        grid=(E // _EBLK,),
        in_specs=in_specs,
        out_specs=out_specs,
        out_shape=outs,
        interpret=_INTERP,
    )(*args)


def _onehot_body(nvoc, t_ref, emb, out):
    oh = (lax.broadcasted_iota(jnp.int32, (t_ref.shape[0], nvoc), 1)
          == t_ref[...]).astype(jnp.float32)
    out[...] = _dot(oh, emb[...])


def _onehot_embed(types, emb):
    E = types.shape[0]
    nvoc = emb.shape[0]
    if nvoc % 8 != 0:
        emb = jnp.pad(emb, ((0, 8 - nvoc % 8), (0, 0)))
        nvoc = emb.shape[0]
    d = emb.shape[1]
    return pl.pallas_call(
        functools.partial(_onehot_body, nvoc),
        grid=(E // _EBLK,),
        in_specs=[_rowspec(_EBLK, 1), _fs(emb.shape)],
        out_specs=_rowspec(_EBLK, d),
        out_shape=jax.ShapeDtypeStruct((E, d), jnp.float32),
        interpret=_INTERP,
    )(types.reshape(E, 1), emb)


def _merge_e_body(e_ref, t_ref, wa, wb, b1, w2, b2, out):
    z = _dot(e_ref[...], wa[...]) + _dot(t_ref[...], wb[...]) + b1[...]
    z = _silu_ln(z)
    z = _dot(z, w2[...]) + b2[...]
    out[...] = _silu_ln(z)


def _merge_e_call(e, t, ps):
    E = e.shape[0]
    w1 = ps[0]['w']
    args = [e, t, w1[:EE], w1[EE:], ps[0]['b'].reshape(1, EE),
            ps[1]['w'], ps[1]['b'].reshape(1, EE)]
    return pl.pallas_call(
        _merge_e_body,
        grid=(E // _EBLK,),
        in_specs=[_rowspec(_EBLK, EE), _rowspec(_EBLK, TD)]
        + [_fs(a.shape) for a in args[2:]],
        out_specs=_rowspec(_EBLK, EE),
        out_shape=jax.ShapeDtypeStruct((E, EE), jnp.float32),
        interpret=_INTERP,
    )(*args)


def _rec_init_body(h_ref, w_ref, out):
    out[...] = _dot(h_ref[...], w_ref[...])


def _rec_init_call(rec_h, w):
    return pl.pallas_call(
        _rec_init_body,
        grid=(N_REC // _NBLK,),
        in_specs=[_rowspec(_NBLK, rec_h.shape[1]), _fs(w.shape)],
        out_specs=_rowspec(_NBLK, H),
        out_shape=jax.ShapeDtypeStruct((N_REC, H), jnp.float32),
        interpret=_INTERP,
    )(rec_h, w)


def _lig_init_body(ty_ref, ba_ref, t_ref, emb, wa, wb, b1, w2, b2,
                   h_out, tb_out):
    blk = ty_ref.shape[0]
    oh = (lax.broadcasted_iota(jnp.int32, (blk, 32), 1)
          == ty_ref[...]).astype(jnp.float32)
    h0 = _dot(oh, emb[...])
    ohb = (lax.broadcasted_iota(jnp.int32, (blk, 8), 1)
           == ba_ref[...]).astype(jnp.float32)
    tb = _dot(ohb, t_ref[...])
    tb_out[...] = tb
    z = _dot(h0, wa[...]) + _dot(tb, wb[...]) + b1[...]
    z = _silu_ln(z)
    z = _dot(z, w2[...]) + b2[...]
    h_out[...] = _silu_ln(z)


def _lig_init_call(lig_h_type, lig_batch, t, emb, ps):
    w1 = ps[0]['w']
    args = [lig_h_type.reshape(N_LIG, 1), lig_batch.reshape(N_LIG, 1), t,
            emb, w1[:H], w1[H:], ps[0]['b'].reshape(1, H),
            ps[1]['w'], ps[1]['b'].reshape(1, H)]
    return pl.pallas_call(
        _lig_init_body,
        grid=(N_LIG // _NBLK,),
        in_specs=[_rowspec(_NBLK, 1), _rowspec(_NBLK, 1)]
        + [_fs(a.shape) for a in args[2:]],
        out_specs=[_rowspec(_NBLK, H), _rowspec(_NBLK, TD)],
        out_shape=[jax.ShapeDtypeStruct((N_LIG, H), jnp.float32),
                   jax.ShapeDtypeStruct((N_LIG, TD), jnp.float32)],
        interpret=_INTERP,
    )(*args)


def _upd_body(nacc, h_ref, wa, wb, b1, w2, b2, *rest):
    accs, out = rest[:nacc], rest[nacc]
    h = h_ref[...]
    agg = accs[0][...][:, :H]
    for a in accs[1:]:
        agg = agg + a[...][:, :H]
    z = _dot(h, wa[...]) + _dot(agg, wb[...]) + b1[...]
    z = _silu_ln(z)
    z = _dot(z, w2[...]) + b2[...]
    out[...] = h + z


def _upd_call(h, accs, ps):
    n = h.shape[0]
    w1 = ps[0]['w']
    flat = []
    for acc in accs:
        for p in range(acc.shape[0]):
            flat.append(acc[p])
    args = [h, w1[:H], w1[H:], ps[0]['b'].reshape(1, HID),
            ps[1]['w'], ps[1]['b'].reshape(1, H)] + flat
    return pl.pallas_call(
        functools.partial(_upd_body, len(flat)),
        grid=(n // _NBLK,),
        in_specs=[_rowspec(_NBLK, H)] + [_fs(a.shape) for a in args[1:6]]
        + [_rowspec(_NBLK, f.shape[1]) for f in flat],
        out_specs=_rowspec(_NBLK, H),
        out_shape=jax.ShapeDtypeStruct((n, H), jnp.float32),
        interpret=_INTERP,
    )(*args)


def _xupd_body(x_ref, *rest):
    accs, out = rest[:-1], rest[-1]
    acc = accs[0][...]
    for a in accs[1:]:
        acc = acc + a[...]
    num = acc[:, HID:HID + XP]
    cnt = acc[:, HID + 3:HID + 4]
    col = lax.broadcasted_iota(jnp.int32, num.shape, 1)
    num = jnp.where(col == 3, 0.0, num)
    out[...] = x_ref[...] + num / (cnt + 1e-8)


def _xupd_call(x, acc):
    n = x.shape[0]
    flat = [acc[p] for p in range(acc.shape[0])]
    return pl.pallas_call(
        _xupd_body,
        grid=(n // _NBLK,),
        in_specs=[_rowspec(_NBLK, XP)]
        + [_rowspec(_NBLK, f.shape[1]) for f in flat],
        out_specs=_rowspec(_NBLK, XP),
        out_shape=jax.ShapeDtypeStruct((n, XP), jnp.float32),
        interpret=_INTERP,
    )(x, *flat)


def _readout_body(h_ref, w1, b1, w2, b2, w3, b3, out):
    z = _dot(h_ref[...], w1[...]) + b1[...]
    z = _silu_ln(z)
    z = _dot(z, w2[...]) + b2[...]
    z = _silu_ln(z)
    out[...] = _dot(z, w3[...]) + b3[...]


def _readout_call(h, ps):
    n = h.shape[0]
    d = ps[2]['w'].shape[1]
    args = [h, ps[0]['w'], ps[0]['b'].reshape(1, -1),
            ps[1]['w'], ps[1]['b'].reshape(1, -1),
            ps[2]['w'], ps[2]['b'].reshape(1, -1)]
    return pl.pallas_call(
        _readout_body,
        grid=(n // _NBLK,),
        in_specs=[_rowspec(_NBLK, H)] + [_fs(a.shape) for a in args[1:]],
        out_specs=_rowspec(_NBLK, d),
        out_shape=jax.ShapeDtypeStruct((n, d), jnp.float32),
        interpret=_INTERP,
    )(*args)


_VMESH = plsc.VectorSubcoreMesh(core_axis_name="c", subcore_axis_name="s")
_W = 80
_NSUB = 16
_NCORE = 2


def _row_share(n, sid):
    per = ((n + _NSUB - 1) // _NSUB + 7) // 8 * 8
    last = n - per * (_NSUB - 1)
    assert last > 0 and last % 8 == 0
    return per, last


def _stage_rows(src, dst, n, sid):
    per, last = _row_share(n, sid)

    @pl.when(sid < _NSUB - 1)
    def _():
        sl = pl.ds(sid * per, per)
        pltpu.sync_copy(src.at[sl], dst.at[sl])

    @pl.when(sid == _NSUB - 1)
    def _():
        sl = pl.ds((_NSUB - 1) * per, last)
        pltpu.sync_copy(src.at[sl], dst.at[sl])


_NW = _NCORE * _NSUB


def _widx(idx):
    nwin = idx.shape[0] // _W
    nk = -(-nwin // _NW)
    idx2 = idx.reshape(nwin, _W)
    idx2 = jnp.pad(idx2, ((0, nk * _NW - nwin), (0, 0)))
    return idx2.reshape(nk, _NW, _W).transpose(1, 0, 2), nwin, nk


def _pipe2(n, start1, wait1, start2, wait2):
    if n > 0:
        start1(0, 0)
    if n > 1:
        start1(1, 1)

    def halfstep(k, b):
        wait1(k, b)
        start2(k, b)

        @pl.when(k + 2 < n)
        def _():
            wait2(k, b)
            start1(k + 2, b)

    @pl.loop(0, n // 2)
    def _(p):
        halfstep(2 * p, 0)
        halfstep(2 * p + 1, 1)

    if n % 2 == 1:
        if n >= 2:
            wait2(n - 2, 1)
        wait1(n - 1, 0)
        start2(n - 1, 0)
        wait2(n - 1, 0)
    else:
        if n >= 2:
            wait2(n - 2, 0)
        if n >= 1:
            wait2(n - 1, 1)


def _gather_rows(table, idx):
    n, D = table.shape
    E = idx.shape[0]
    idx3, nwin, nk = _widx(idx)
    n_main = nwin // _NW
    rem = nwin % _NW

    @functools.partial(
        pl.kernel,
        out_type=jax.ShapeDtypeStruct((E, D), jnp.float32),
        mesh=_VMESH,
        compiler_params=pltpu.CompilerParams(use_tc_tiling_on_sc=False),
        scratch_types=[pltpu.VMEM((nk, _W), jnp.int32),
                       pltpu.VMEM((_W, D), jnp.float32),
                       pltpu.VMEM((_W, D), jnp.float32),
                       pltpu.VMEM_SHARED((n, D), jnp.float32),
                       pltpu.SemaphoreType.DMA,
                       pltpu.SemaphoreType.DMA,
                       pltpu.SemaphoreType.DMA,
                       pltpu.SemaphoreType.DMA],
        name=f"sc_gather_{E}_{D}",
    )
    def k(table_hbm, idx_hbm, out_hbm, i_all, rows_a, rows_b, tab_sh,
          s1a, s1b, s2a, s2b):
        cid = lax.axis_index("c")
        sid = lax.axis_index("s")
        wid = sid * _NCORE + cid
        bufs = (rows_a, rows_b)
        s1 = (s1a, s1b)
        s2 = (s2a, s2b)

        pltpu.sync_copy(idx_hbm.at[wid], i_all)
        _stage_rows(table_hbm, tab_sh, n, sid)
        plsc.subcore_barrier()

        def start1(kk, b):
            pltpu.async_copy(tab_sh.at[i_all.at[kk]], bufs[b], s1[b])

        def wait1(kk, b):
            pltpu.make_async_copy(tab_sh.at[i_all.at[kk]], bufs[b],
                                  s1[b]).wait()

        def _out(kk):
            return out_hbm.at[pl.ds((kk * _NW + wid) * _W, _W)]

        def start2(kk, b):
            pltpu.async_copy(bufs[b], _out(kk), s2[b])

        def wait2(kk, b):
            pltpu.make_async_copy(bufs[b], _out(kk), s2[b]).wait()

        _pipe2(n_main, start1, wait1, start2, wait2)

        if rem:
            @pl.when(wid < rem)
            def _():
                pltpu.sync_copy(tab_sh.at[i_all.at[n_main]], rows_a)
                pltpu.sync_copy(rows_a, _out(n_main))

    return k(table, idx3)


def _scatter_add(pairs, n):
    D = pairs[0][0].shape[1]
    flat = []
    meta = []
    nk_max = 0
    for v, idx in pairs:
        idx3, nwin, nk = _widx(idx)
        flat += [v, idx3]
        meta.append((nwin // _NW, nwin % _NW, nk))
        nk_max = max(nk_max, nk)
    zeros = jnp.zeros((n, D), jnp.float32)
    tag = "_".join(str(v.shape[0]) for v, _ in pairs)

    @functools.partial(
        pl.kernel,
        out_type=jax.ShapeDtypeStruct((_NCORE, n, D), jnp.float32),
        mesh=_VMESH,
        compiler_params=pltpu.CompilerParams(use_tc_tiling_on_sc=False),
        scratch_types=[pltpu.VMEM((nk_max, _W), jnp.int32),
                       pltpu.VMEM((_W, D), jnp.float32),
                       pltpu.VMEM((_W, D), jnp.float32),
                       pltpu.VMEM_SHARED((n, D), jnp.float32),
                       pltpu.SemaphoreType.DMA,
                       pltpu.SemaphoreType.DMA,
                       pltpu.SemaphoreType.DMA,
                       pltpu.SemaphoreType.DMA],
        name=f"sc_scatter_{tag}_{D}",
    )
    def k(*refs):
        *ins, out_hbm, i_all, v_a, v_b, acc_sh, s1a, s1b, s2a, s2b = refs
        zeros_hbm = ins[-1]
        cid = lax.axis_index("c")
        sid = lax.axis_index("s")
        wid = sid * _NCORE + cid
        bufs = (v_a, v_b)
        s1 = (s1a, s1b)
        s2 = (s2a, s2b)

        _stage_rows(zeros_hbm, acc_sh, n, sid)
        plsc.subcore_barrier()

        for p, (n_main, rem, nk) in enumerate(meta):
            vals_hbm = ins[2 * p]
            idx_hbm = ins[2 * p + 1]
            pltpu.sync_copy(idx_hbm.at[wid], i_all.at[pl.ds(0, nk)])

            def _src(kk, vals_hbm=vals_hbm):
                return vals_hbm.at[pl.ds((kk * _NW + wid) * _W, _W)]

            def start1(kk, b):
                pltpu.async_copy(_src(kk), bufs[b], s1[b])

            def wait1(kk, b):
                pltpu.make_async_copy(_src(kk), bufs[b], s1[b]).wait()

            def start2(kk, b):
                pltpu.async_copy(bufs[b], acc_sh.at[i_all.at[kk]], s2[b],
                                 add=True)

            def wait2(kk, b):
                pltpu.make_async_copy(bufs[b], acc_sh.at[i_all.at[kk]],
                                      s2[b]).wait()

            _pipe2(n_main, start1, wait1, start2, wait2)

            if rem:
                @pl.when(wid < rem)
                def _():
                    pltpu.sync_copy(_src(n_main), v_a)
                    pltpu.sync_copy(v_a, acc_sh.at[i_all.at[n_main]],
                                    add=True)

        plsc.subcore_barrier()
        _stage_rows(acc_sh, out_hbm.at[cid], n, sid)

    return k(*flat, zeros)


def kernel(rec_h, rec_x, rec_e_index, rec_e_type, rec_batch, lig_h_type,
           lig_x, lig_e_index, lig_e_type, lig_batch, timestep,
           inter_e_index, inter_e_type, params):
    t = jnp.asarray(_PE)[timestep]
    for i, p in enumerate(params['embd_timestep']):
        t = t @ p['w'] + p['b']
        t = jax.nn.silu(_ln(t))

    r0, r1 = rec_e_index[0], rec_e_index[1]
    l0, l1 = lig_e_index[0], lig_e_index[1]
    isrc, idst = inter_e_index[0], inter_e_index[1]

    rec_xp = jnp.pad(rec_x, ((0, 0), (0, XP - 3)))
    lig_xp = jnp.pad(lig_x, ((0, 0), (0, XP - 3)))

    rec_h_cur = _rec_init_call(rec_h, params['embd_rec_h'])
    lig_h, tB = _lig_init_call(lig_h_type, lig_batch, t,
                               params['embd_lig_h'], params['merge_lig_h'])

    rec_e_raw = _onehot_embed(rec_e_type, params['embd_rec_e'])
    re = [rec_e_raw, rec_e_raw]
    lig_e_raw = _onehot_embed(lig_e_type, params['embd_lig_e'])
    inter_e_raw = _onehot_embed(inter_e_type, params['embd_inter_e'])
    tb0 = _gather_rows(tB, l0)
    tb1 = _gather_rows(tB, l1)
    le = [_merge_e_call(lig_e_raw, tb1, params['merge_lig_e']),
          _merge_e_call(lig_e_raw, tb0, params['merge_lig_e'])]
    t_inter_e = _gather_rows(tB, idst)
    inter_e = _merge_e_call(inter_e_raw, t_inter_e, params['merge_inter_e'])

    for lp in params['layers']:
        tab_r = jnp.concatenate([rec_h_cur, rec_xp], axis=1)
        g0 = _gather_rows(tab_r, r0)
        g1 = _gather_rows(tab_r, r1)
        m0, re0 = _msg_call(g0, g1, re[0], lp['rr_msg'], lp['rr_e'])
        m1, re1 = _msg_call(g1, g0, re[1], lp['rr_msg'], lp['rr_e'])
        re = [re0, re1]
        acc_rr = _scatter_add([(m0, r1), (m1, r0)], N_REC)
        rec_h_cur = _upd_call(rec_h_cur, [acc_rr], lp['rr_upd'])

        tab_l = jnp.concatenate([lig_h, lig_xp], axis=1)
        g0 = _gather_rows(tab_l, l0)
        g1 = _gather_rows(tab_l, l1)
        msc0, le0 = _msg_call(g0, g1, le[0], lp['ll_msg'], lp['ll_e'],
                              lp['ll_x'])
        msc1, le1 = _msg_call(g1, g0, le[1], lp['ll_msg'], lp['ll_e'],
                              lp['ll_x'])
        le = [le0, le1]
        acc_ll = _scatter_add([(msc0, l1), (msc1, l0)], N_LIG)
        lig_xp = _xupd_call(lig_xp, acc_ll)

        tab_r2 = jnp.concatenate([rec_h_cur, rec_xp], axis=1)
        tab_l2 = jnp.concatenate([lig_h, lig_xp], axis=1)
        gr = _gather_rows(tab_r2, isrc)
        gl = _gather_rows(tab_l2, idst)
        msc_rl, inter_e = _msg_call(gr, gl, inter_e,
                                    lp['rl_msg'], lp['rl_e'], lp['rl_x'])
        acc_rl = _scatter_add([(msc_rl, idst)], N_LIG)
        lig_xp = _xupd_call(lig_xp, acc_rl)
        lig_h = _upd_call(lig_h, [acc_ll, acc_rl], lp['ll_upd'])

    lig_h_out = _readout_call(lig_h, params['readout_lig_h'])
    return lig_h_out, lig_xp[:, :3]


# speedup vs baseline: 3.8741x; 1.0547x over previous
import functools

import jax
import jax.numpy as jnp
import numpy as np
from jax import lax
from jax.experimental import pallas as pl
from jax.experimental.pallas import tpu as pltpu
from jax.experimental.pallas import tpu_sc as plsc

N_REC = 10000
N_LIG = 10000
H = 128
EE = 64
HID = 128
TD = 128
NT = 1000
XP = 16

_EBLK = 4000
_NBLK = 1000

_INTERP = False


def _pe_table_np(d, n):
    pos = np.arange(n)[:, None].astype(np.float64)
    i = np.arange(d)[None, :]
    ang = pos / np.power(10000.0, (2 * (i // 2)) / d)
    t = np.zeros((n, d))
    t[:, 0::2] = np.sin(ang[:, 0::2])
    t[:, 1::2] = np.cos(ang[:, 1::2])
    return t.astype(np.float32)


_PE = _pe_table_np(TD, NT)


def _ln(x):
    m = jnp.mean(x, axis=-1, keepdims=True)
    v = jnp.mean((x - m) * (x - m), axis=-1, keepdims=True)
    return (x - m) * lax.rsqrt(v + 1e-5)


def _silu_ln(x):
    y = _ln(x)
    return y * jax.nn.sigmoid(y)


def _fs(shape):
    nd = len(shape)
    return pl.BlockSpec(shape, lambda i, _n=nd: (0,) * _n)


def _rowspec(blk, width):
    return pl.BlockSpec((blk, width), lambda i: (i, 0))


def _dot(a, b):
    return jnp.dot(a, b, preferred_element_type=jnp.float32)


def _msg_body(has_x, gs_ref, gd_ref, e_ref,
              w1hs, w1hd, w1e, w1d, b1, w2, b2, we, be, *rest):
    if has_x:
        wx, bx, msc_out, e_out = rest
    else:
        msc_out, e_out = rest
    gs = gs_ref[...]
    gd = gd_ref[...]
    hs = gs[:, :H]
    hd = gd[:, :H]
    e = e_ref[...]
    diff = gd[:, H:] - gs[:, H:]
    d2 = jnp.sum(diff * diff, axis=1, keepdims=True)
    z = (_dot(hs, w1hs[...]) + _dot(hd, w1hd[...]) + _dot(e, w1e[...])
         + d2 * w1d[...] + b1[...])
    z = _silu_ln(z)
    z = _dot(z, w2[...]) + b2[...]
    m = _silu_ln(z)
    e_out[...] = e + _dot(m, we[...]) + be[...]
    if has_x:
        coef = jnp.sum(m * wx[...], axis=1, keepdims=True) + bx[...]
        sc = diff * coef
        col = lax.broadcasted_iota(jnp.int32, sc.shape, 1)
        sc = jnp.where(col == 3, 1.0, sc)
        msc_out[...] = jnp.concatenate([m, sc], axis=1)
    else:
        msc_out[...] = m


def _msg_call(gs, gd, e, msg_ps, e_ps, x_ps=None):
    E = gs.shape[0]
    has_x = x_ps is not None
    w1 = msg_ps[0]['w']
    w1hs = w1[:H]
    w1hd = w1[H:2 * H]
    w1e = w1[2 * H:2 * H + EE]
    w1d = w1[2 * H + EE:].reshape(1, HID)
    b1 = msg_ps[0]['b'].reshape(1, HID)
    w2 = msg_ps[1]['w']
    b2 = msg_ps[1]['b'].reshape(1, HID)
    we = e_ps['w']
    be = e_ps['b'].reshape(1, EE)
    args = [gs, gd, e, w1hs, w1hd, w1e, w1d, b1, w2, b2, we, be]
    mw = HID + XP if has_x else HID
    outs = [jax.ShapeDtypeStruct((E, mw), jnp.float32),
            jax.ShapeDtypeStruct((E, EE), jnp.float32)]
    out_specs = [_rowspec(_EBLK, mw), _rowspec(_EBLK, EE)]
    if has_x:
        args += [x_ps['w'].reshape(1, HID), x_ps['b'].reshape(1, 1)]
    in_specs = [_rowspec(_EBLK, H + XP), _rowspec(_EBLK, H + XP),
                _rowspec(_EBLK, EE)]
    in_specs += [_fs(a.shape) for a in args[3:]]
    return pl.pallas_call(
        functools.partial(_msg_body, has_x),
        grid=(E // _EBLK,),
        in_specs=in_specs,
        out_specs=out_specs,
        out_shape=outs,
        interpret=_INTERP,
    )(*args)


def _onehot_body(nvoc, t_ref, emb, out):
    oh = (lax.broadcasted_iota(jnp.int32, (t_ref.shape[0], nvoc), 1)
          == t_ref[...]).astype(jnp.float32)
    out[...] = _dot(oh, emb[...])


def _onehot_embed(types, emb):
    E = types.shape[0]
    nvoc = emb.shape[0]
    if nvoc % 8 != 0:
        emb = jnp.pad(emb, ((0, 8 - nvoc % 8), (0, 0)))
        nvoc = emb.shape[0]
    d = emb.shape[1]
    return pl.pallas_call(
        functools.partial(_onehot_body, nvoc),
        grid=(E // _EBLK,),
        in_specs=[_rowspec(_EBLK, 1), _fs(emb.shape)],
        out_specs=_rowspec(_EBLK, d),
        out_shape=jax.ShapeDtypeStruct((E, d), jnp.float32),
        interpret=_INTERP,
    )(types.reshape(E, 1), emb)


def _merge_e_body(e_ref, t_ref, wa, wb, b1, w2, b2, out):
    z = _dot(e_ref[...], wa[...]) + _dot(t_ref[...], wb[...]) + b1[...]
    z = _silu_ln(z)
    z = _dot(z, w2[...]) + b2[...]
    out[...] = _silu_ln(z)


def _merge_e_call(e, t, ps):
    E = e.shape[0]
    w1 = ps[0]['w']
    args = [e, t, w1[:EE], w1[EE:], ps[0]['b'].reshape(1, EE),
            ps[1]['w'], ps[1]['b'].reshape(1, EE)]
    return pl.pallas_call(
        _merge_e_body,
        grid=(E // _EBLK,),
        in_specs=[_rowspec(_EBLK, EE), _rowspec(_EBLK, TD)]
        + [_fs(a.shape) for a in args[2:]],
        out_specs=_rowspec(_EBLK, EE),
        out_shape=jax.ShapeDtypeStruct((E, EE), jnp.float32),
        interpret=_INTERP,
    )(*args)


def _rec_init_body(h_ref, w_ref, out):
    out[...] = _dot(h_ref[...], w_ref[...])


def _rec_init_call(rec_h, w):
    return pl.pallas_call(
        _rec_init_body,
        grid=(N_REC // _NBLK,),
        in_specs=[_rowspec(_NBLK, rec_h.shape[1]), _fs(w.shape)],
        out_specs=_rowspec(_NBLK, H),
        out_shape=jax.ShapeDtypeStruct((N_REC, H), jnp.float32),
        interpret=_INTERP,
    )(rec_h, w)


def _lig_init_body(ty_ref, ba_ref, t_ref, emb, wa, wb, b1, w2, b2,
                   h_out, tb_out):
    blk = ty_ref.shape[0]
    oh = (lax.broadcasted_iota(jnp.int32, (blk, 32), 1)
          == ty_ref[...]).astype(jnp.float32)
    h0 = _dot(oh, emb[...])
    ohb = (lax.broadcasted_iota(jnp.int32, (blk, 8), 1)
           == ba_ref[...]).astype(jnp.float32)
    tb = _dot(ohb, t_ref[...])
    tb_out[...] = tb
    z = _dot(h0, wa[...]) + _dot(tb, wb[...]) + b1[...]
    z = _silu_ln(z)
    z = _dot(z, w2[...]) + b2[...]
    h_out[...] = _silu_ln(z)


def _lig_init_call(lig_h_type, lig_batch, t, emb, ps):
    w1 = ps[0]['w']
    args = [lig_h_type.reshape(N_LIG, 1), lig_batch.reshape(N_LIG, 1), t,
            emb, w1[:H], w1[H:], ps[0]['b'].reshape(1, H),
            ps[1]['w'], ps[1]['b'].reshape(1, H)]
    return pl.pallas_call(
        _lig_init_body,
        grid=(N_LIG // _NBLK,),
        in_specs=[_rowspec(_NBLK, 1), _rowspec(_NBLK, 1)]
        + [_fs(a.shape) for a in args[2:]],
        out_specs=[_rowspec(_NBLK, H), _rowspec(_NBLK, TD)],
        out_shape=[jax.ShapeDtypeStruct((N_LIG, H), jnp.float32),
                   jax.ShapeDtypeStruct((N_LIG, TD), jnp.float32)],
        interpret=_INTERP,
    )(*args)


def _upd_body(nacc, h_ref, wa, wb, b1, w2, b2, *rest):
    accs, out = rest[:nacc], rest[nacc]
    h = h_ref[...]
    agg = accs[0][...][:, :H]
    for a in accs[1:]:
        agg = agg + a[...][:, :H]
    z = _dot(h, wa[...]) + _dot(agg, wb[...]) + b1[...]
    z = _silu_ln(z)
    z = _dot(z, w2[...]) + b2[...]
    out[...] = h + z


def _upd_call(h, accs, ps):
    n = h.shape[0]
    w1 = ps[0]['w']
    flat = []
    for acc in accs:
        for p in range(acc.shape[0]):
            flat.append(acc[p])
    args = [h, w1[:H], w1[H:], ps[0]['b'].reshape(1, HID),
            ps[1]['w'], ps[1]['b'].reshape(1, H)] + flat
    return pl.pallas_call(
        functools.partial(_upd_body, len(flat)),
        grid=(n // _NBLK,),
        in_specs=[_rowspec(_NBLK, H)] + [_fs(a.shape) for a in args[1:6]]
        + [_rowspec(_NBLK, f.shape[1]) for f in flat],
        out_specs=_rowspec(_NBLK, H),
        out_shape=jax.ShapeDtypeStruct((n, H), jnp.float32),
        interpret=_INTERP,
    )(*args)


def _xupd_body(x_ref, *rest):
    accs, out = rest[:-1], rest[-1]
    acc = accs[0][...]
    for a in accs[1:]:
        acc = acc + a[...]
    num = acc[:, HID:HID + XP]
    cnt = acc[:, HID + 3:HID + 4]
    col = lax.broadcasted_iota(jnp.int32, num.shape, 1)
    num = jnp.where(col == 3, 0.0, num)
    out[...] = x_ref[...] + num / (cnt + 1e-8)


def _xupd_call(x, acc):
    n = x.shape[0]
    flat = [acc[p] for p in range(acc.shape[0])]
    return pl.pallas_call(
        _xupd_body,
        grid=(n // _NBLK,),
        in_specs=[_rowspec(_NBLK, XP)]
        + [_rowspec(_NBLK, f.shape[1]) for f in flat],
        out_specs=_rowspec(_NBLK, XP),
        out_shape=jax.ShapeDtypeStruct((n, XP), jnp.float32),
        interpret=_INTERP,
    )(x, *flat)


def _readout_body(h_ref, w1, b1, w2, b2, w3, b3, out):
    z = _dot(h_ref[...], w1[...]) + b1[...]
    z = _silu_ln(z)
    z = _dot(z, w2[...]) + b2[...]
    z = _silu_ln(z)
    out[...] = _dot(z, w3[...]) + b3[...]


def _readout_call(h, ps):
    n = h.shape[0]
    d = ps[2]['w'].shape[1]
    args = [h, ps[0]['w'], ps[0]['b'].reshape(1, -1),
            ps[1]['w'], ps[1]['b'].reshape(1, -1),
            ps[2]['w'], ps[2]['b'].reshape(1, -1)]
    return pl.pallas_call(
        _readout_body,
        grid=(n // _NBLK,),
        in_specs=[_rowspec(_NBLK, H)] + [_fs(a.shape) for a in args[1:]],
        out_specs=_rowspec(_NBLK, d),
        out_shape=jax.ShapeDtypeStruct((n, d), jnp.float32),
        interpret=_INTERP,
    )(*args)


_VMESH = plsc.VectorSubcoreMesh(core_axis_name="c", subcore_axis_name="s")
_W = 80
_NSUB = 16
_NCORE = 2


def _row_share(n, sid):
    per = ((n + _NSUB - 1) // _NSUB + 7) // 8 * 8
    last = n - per * (_NSUB - 1)
    assert last > 0 and last % 8 == 0
    return per, last


def _stage_rows(src, dst, n, sid):
    per, last = _row_share(n, sid)

    @pl.when(sid < _NSUB - 1)
    def _():
        sl = pl.ds(sid * per, per)
        pltpu.sync_copy(src.at[sl], dst.at[sl])

    @pl.when(sid == _NSUB - 1)
    def _():
        sl = pl.ds((_NSUB - 1) * per, last)
        pltpu.sync_copy(src.at[sl], dst.at[sl])


_NW = _NCORE * _NSUB


def _widx(idx):
    nwin = idx.shape[0] // _W
    nk = -(-nwin // _NW)
    idx2 = idx.reshape(nwin, _W)
    idx2 = jnp.pad(idx2, ((0, nk * _NW - nwin), (0, 0)))
    return idx2.reshape(nk, _NW, _W).transpose(1, 0, 2), nwin, nk


def _pipe2(n, start1, wait1, start2, wait2):
    if n > 0:
        start1(0, 0)
    if n > 1:
        start1(1, 1)

    def halfstep(k, b):
        wait1(k, b)
        start2(k, b)

        @pl.when(k + 2 < n)
        def _():
            wait2(k, b)
            start1(k + 2, b)

    @pl.loop(0, n // 2)
    def _(p):
        halfstep(2 * p, 0)
        halfstep(2 * p + 1, 1)

    if n % 2 == 1:
        if n >= 2:
            wait2(n - 2, 1)
        wait1(n - 1, 0)
        start2(n - 1, 0)
        wait2(n - 1, 0)
    else:
        if n >= 2:
            wait2(n - 2, 0)
        if n >= 1:
            wait2(n - 1, 1)


def _gather_rows(table, idx):
    n, D = table.shape
    E = idx.shape[0]
    idx3, nwin, nk = _widx(idx)
    n_main = nwin // _NW
    rem = nwin % _NW

    @functools.partial(
        pl.kernel,
        out_type=jax.ShapeDtypeStruct((E, D), jnp.float32),
        mesh=_VMESH,
        compiler_params=pltpu.CompilerParams(use_tc_tiling_on_sc=False),
        scratch_types=[pltpu.VMEM((nk, _W), jnp.int32),
                       pltpu.VMEM((_W, D), jnp.float32),
                       pltpu.VMEM((_W, D), jnp.float32),
                       pltpu.VMEM_SHARED((n, D), jnp.float32),
                       pltpu.SemaphoreType.DMA,
                       pltpu.SemaphoreType.DMA,
                       pltpu.SemaphoreType.DMA,
                       pltpu.SemaphoreType.DMA],
        name=f"sc_gather_{E}_{D}",
    )
    def k(table_hbm, idx_hbm, out_hbm, i_all, rows_a, rows_b, tab_sh,
          s1a, s1b, s2a, s2b):
        cid = lax.axis_index("c")
        sid = lax.axis_index("s")
        wid = sid * _NCORE + cid
        bufs = (rows_a, rows_b)
        s1 = (s1a, s1b)
        s2 = (s2a, s2b)

        pltpu.sync_copy(idx_hbm.at[wid], i_all)
        _stage_rows(table_hbm, tab_sh, n, sid)
        plsc.subcore_barrier()

        def start1(kk, b):
            pltpu.async_copy(tab_sh.at[i_all.at[kk]], bufs[b], s1[b])

        def wait1(kk, b):
            pltpu.make_async_copy(tab_sh.at[i_all.at[kk]], bufs[b],
                                  s1[b]).wait()

        def _out(kk):
            return out_hbm.at[pl.ds((kk * _NW + wid) * _W, _W)]

        def start2(kk, b):
            pltpu.async_copy(bufs[b], _out(kk), s2[b])

        def wait2(kk, b):
            pltpu.make_async_copy(bufs[b], _out(kk), s2[b]).wait()

        _pipe2(n_main, start1, wait1, start2, wait2)

        if rem:
            @pl.when(wid < rem)
            def _():
                pltpu.sync_copy(tab_sh.at[i_all.at[n_main]], rows_a)
                pltpu.sync_copy(rows_a, _out(n_main))

    return k(table, idx3)


def _scatter_add(pairs, n):
    D = pairs[0][0].shape[1]
    flat = []
    meta = []
    nk_max = 0
    for v, idx in pairs:
        idx3, nwin, nk = _widx(idx)
        flat += [v, idx3]
        meta.append((nwin // _NW, nwin % _NW, nk))
        nk_max = max(nk_max, nk)
    zeros = jnp.zeros((n, D), jnp.float32)
    tag = "_".join(str(v.shape[0]) for v, _ in pairs)

    @functools.partial(
        pl.kernel,
        out_type=jax.ShapeDtypeStruct((_NCORE, n, D), jnp.float32),
        mesh=_VMESH,
        compiler_params=pltpu.CompilerParams(use_tc_tiling_on_sc=False),
        scratch_types=[pltpu.VMEM((nk_max, _W), jnp.int32),
                       pltpu.VMEM((_W, D), jnp.float32),
                       pltpu.VMEM((_W, D), jnp.float32),
                       pltpu.VMEM_SHARED((n, D), jnp.float32),
                       pltpu.SemaphoreType.DMA,
                       pltpu.SemaphoreType.DMA,
                       pltpu.SemaphoreType.DMA,
                       pltpu.SemaphoreType.DMA],
        name=f"sc_scatter_{tag}_{D}",
    )
    def k(*refs):
        *ins, out_hbm, i_all, v_a, v_b, acc_sh, s1a, s1b, s2a, s2b = refs
        zeros_hbm = ins[-1]
        cid = lax.axis_index("c")
        sid = lax.axis_index("s")
        wid = sid * _NCORE + cid
        bufs = (v_a, v_b)
        s1 = (s1a, s1b)
        s2 = (s2a, s2b)

        _stage_rows(zeros_hbm, acc_sh, n, sid)
        plsc.subcore_barrier()

        for p, (n_main, rem, nk) in enumerate(meta):
            vals_hbm = ins[2 * p]
            idx_hbm = ins[2 * p + 1]
            pltpu.sync_copy(idx_hbm.at[wid], i_all.at[pl.ds(0, nk)])

            def _src(kk, vals_hbm=vals_hbm):
                return vals_hbm.at[pl.ds((kk * _NW + wid) * _W, _W)]

            def start1(kk, b):
                pltpu.async_copy(_src(kk), bufs[b], s1[b])

            def wait1(kk, b):
                pltpu.make_async_copy(_src(kk), bufs[b], s1[b]).wait()

            def start2(kk, b):
                pltpu.async_copy(bufs[b], acc_sh.at[i_all.at[kk]], s2[b],
                                 add=True)

            def wait2(kk, b):
                pltpu.make_async_copy(bufs[b], acc_sh.at[i_all.at[kk]],
                                      s2[b]).wait()

            _pipe2(n_main, start1, wait1, start2, wait2)

            if rem:
                @pl.when(wid < rem)
                def _():
                    pltpu.sync_copy(_src(n_main), v_a)
                    pltpu.sync_copy(v_a, acc_sh.at[i_all.at[n_main]],
                                    add=True)

        plsc.subcore_barrier()
        _stage_rows(acc_sh, out_hbm.at[cid], n, sid)

    return k(*flat, zeros)


def kernel(rec_h, rec_x, rec_e_index, rec_e_type, rec_batch, lig_h_type,
           lig_x, lig_e_index, lig_e_type, lig_batch, timestep,
           inter_e_index, inter_e_type, params):
    t = jnp.asarray(_PE)[timestep]
    for i, p in enumerate(params['embd_timestep']):
        t = t @ p['w'] + p['b']
        t = jax.nn.silu(_ln(t))

    r0, r1 = rec_e_index[0], rec_e_index[1]
    l0, l1 = lig_e_index[0], lig_e_index[1]
    isrc, idst = inter_e_index[0], inter_e_index[1]

    rec_xp = jnp.pad(rec_x, ((0, 0), (0, XP - 3)))
    lig_xp = jnp.pad(lig_x, ((0, 0), (0, XP - 3)))

    rec_h_cur = _rec_init_call(rec_h, params['embd_rec_h'])
    lig_h, tB = _lig_init_call(lig_h_type, lig_batch, t,
                               params['embd_lig_h'], params['merge_lig_h'])

    rec_e_raw = _onehot_embed(rec_e_type, params['embd_rec_e'])
    re = [rec_e_raw, rec_e_raw]
    lig_e_raw = _onehot_embed(lig_e_type, params['embd_lig_e'])
    inter_e_raw = _onehot_embed(inter_e_type, params['embd_inter_e'])
    tb0 = _gather_rows(tB, l0)
    tb1 = _gather_rows(tB, l1)
    le = [_merge_e_call(lig_e_raw, tb1, params['merge_lig_e']),
          _merge_e_call(lig_e_raw, tb0, params['merge_lig_e'])]
    t_inter_e = _gather_rows(tB, idst)
    inter_e = _merge_e_call(inter_e_raw, t_inter_e, params['merge_inter_e'])

    for lp in params['layers']:
        tab_r = jnp.concatenate([rec_h_cur, rec_xp], axis=1)
        g0 = _gather_rows(tab_r, r0)
        g1 = _gather_rows(tab_r, r1)
        m0, re0 = _msg_call(g0, g1, re[0], lp['rr_msg'], lp['rr_e'])
        m1, re1 = _msg_call(g1, g0, re[1], lp['rr_msg'], lp['rr_e'])
        re = [re0, re1]
        acc_rr = _scatter_add([(m0, r1), (m1, r0)], N_REC)
        rec_h_cur = _upd_call(rec_h_cur, [acc_rr], lp['rr_upd'])

        tab_l = jnp.concatenate([lig_h, lig_xp], axis=1)
        g0 = _gather_rows(tab_l, l0)
        g1 = _gather_rows(tab_l, l1)
        msc0, le0 = _msg_call(g0, g1, le[0], lp['ll_msg'], lp['ll_e'],
                              lp['ll_x'])
        msc1, le1 = _msg_call(g1, g0, le[1], lp['ll_msg'], lp['ll_e'],
                              lp['ll_x'])
        le = [le0, le1]
        acc_ll = _scatter_add([(msc0, l1), (msc1, l0)], N_LIG)
        lig_xp = _xupd_call(lig_xp, acc_ll)

        tab_r2 = jnp.concatenate([rec_h_cur, rec_xp], axis=1)
        tab_l2 = jnp.concatenate([lig_h, lig_xp], axis=1)
        gr = _gather_rows(tab_r2, isrc)
        gl = _gather_rows(tab_l2, idst)
        msc_rl, inter_e = _msg_call(gr, gl, inter_e,
                                    lp['rl_msg'], lp['rl_e'], lp['rl_x'])
        acc_rl = _scatter_add([(msc_rl, idst)], N_LIG)
        lig_xp = _xupd_call(lig_xp, acc_rl)
        lig_h = _upd_call(lig_h, [acc_ll, acc_rl], lp['ll_upd'])

    lig_h_out = _readout_call(lig_h, params['readout_lig_h'])
    return lig_h_out, lig_xp[:, :3]


# speedup vs baseline: 3.8840x; 1.0026x over previous
import functools

import jax
import jax.numpy as jnp
import numpy as np
from jax import lax
from jax.experimental import pallas as pl
from jax.experimental.pallas import tpu as pltpu
from jax.experimental.pallas import tpu_sc as plsc

N_REC = 10000
N_LIG = 10000
H = 128
EE = 64
HID = 128
TD = 128
NT = 1000
XP = 16

_EBLK = 4000
_NBLK = 2000

_INTERP = False


def _pe_table_np(d, n):
    pos = np.arange(n)[:, None].astype(np.float64)
    i = np.arange(d)[None, :]
    ang = pos / np.power(10000.0, (2 * (i // 2)) / d)
    t = np.zeros((n, d))
    t[:, 0::2] = np.sin(ang[:, 0::2])
    t[:, 1::2] = np.cos(ang[:, 1::2])
    return t.astype(np.float32)


_PE = _pe_table_np(TD, NT)


def _ln(x):
    m = jnp.mean(x, axis=-1, keepdims=True)
    v = jnp.mean((x - m) * (x - m), axis=-1, keepdims=True)
    return (x - m) * lax.rsqrt(v + 1e-5)


def _silu_ln(x):
    y = _ln(x)
    return y * jax.nn.sigmoid(y)


def _fs(shape):
    nd = len(shape)
    return pl.BlockSpec(shape, lambda i, _n=nd: (0,) * _n)


def _rowspec(blk, width):
    return pl.BlockSpec((blk, width), lambda i: (i, 0))


def _dot(a, b):
    return jnp.dot(a, b, preferred_element_type=jnp.float32)


def _msg_body(has_x, gs_ref, gd_ref, e_ref,
              w1hs, w1hd, w1e, w1d, b1, w2, b2, we, be, *rest):
    if has_x:
        wx, bx, msc_out, e_out = rest
    else:
        msc_out, e_out = rest
    gs = gs_ref[...]
    gd = gd_ref[...]
    hs = gs[:, :H]
    hd = gd[:, :H]
    e = e_ref[...]
    diff = gd[:, H:] - gs[:, H:]
    d2 = jnp.sum(diff * diff, axis=1, keepdims=True)
    z = (_dot(hs, w1hs[...]) + _dot(hd, w1hd[...]) + _dot(e, w1e[...])
         + d2 * w1d[...] + b1[...])
    z = _silu_ln(z)
    z = _dot(z, w2[...]) + b2[...]
    m = _silu_ln(z)
    e_out[...] = e + _dot(m, we[...]) + be[...]
    if has_x:
        coef = jnp.sum(m * wx[...], axis=1, keepdims=True) + bx[...]
        sc = diff * coef
        col = lax.broadcasted_iota(jnp.int32, sc.shape, 1)
        sc = jnp.where(col == 3, 1.0, sc)
        msc_out[...] = jnp.concatenate([m, sc], axis=1)
    else:
        msc_out[...] = m


def _msg_call(gs, gd, e, msg_ps, e_ps, x_ps=None):
    E = gs.shape[0]
    has_x = x_ps is not None
    w1 = msg_ps[0]['w']
    w1hs = w1[:H]
    w1hd = w1[H:2 * H]
    w1e = w1[2 * H:2 * H + EE]
    w1d = w1[2 * H + EE:].reshape(1, HID)
    b1 = msg_ps[0]['b'].reshape(1, HID)
    w2 = msg_ps[1]['w']
    b2 = msg_ps[1]['b'].reshape(1, HID)
    we = e_ps['w']
    be = e_ps['b'].reshape(1, EE)
    args = [gs, gd, e, w1hs, w1hd, w1e, w1d, b1, w2, b2, we, be]
    mw = HID + XP if has_x else HID
    outs = [jax.ShapeDtypeStruct((E, mw), jnp.float32),
            jax.ShapeDtypeStruct((E, EE), jnp.float32)]
    out_specs = [_rowspec(_EBLK, mw), _rowspec(_EBLK, EE)]
    if has_x:
        args += [x_ps['w'].reshape(1, HID), x_ps['b'].reshape(1, 1)]
    in_specs = [_rowspec(_EBLK, H + XP), _rowspec(_EBLK, H + XP),
                _rowspec(_EBLK, EE)]
    in_specs += [_fs(a.shape) for a in args[3:]]
    return pl.pallas_call(
        functools.partial(_msg_body, has_x),
        grid=(E // _EBLK,),
        in_specs=in_specs,
        out_specs=out_specs,
        out_shape=outs,
        interpret=_INTERP,
    )(*args)


def _onehot_body(nvoc, t_ref, emb, out):
    oh = (lax.broadcasted_iota(jnp.int32, (t_ref.shape[0], nvoc), 1)
          == t_ref[...]).astype(jnp.float32)
    out[...] = _dot(oh, emb[...])


def _onehot_embed(types, emb):
    E = types.shape[0]
    nvoc = emb.shape[0]
    if nvoc % 8 != 0:
        emb = jnp.pad(emb, ((0, 8 - nvoc % 8), (0, 0)))
        nvoc = emb.shape[0]
    d = emb.shape[1]
    return pl.pallas_call(
        functools.partial(_onehot_body, nvoc),
        grid=(E // _EBLK,),
        in_specs=[_rowspec(_EBLK, 1), _fs(emb.shape)],
        out_specs=_rowspec(_EBLK, d),
        out_shape=jax.ShapeDtypeStruct((E, d), jnp.float32),
        interpret=_INTERP,
    )(types.reshape(E, 1), emb)


def _merge_e_body(e_ref, t_ref, wa, wb, b1, w2, b2, out):
    z = _dot(e_ref[...], wa[...]) + _dot(t_ref[...], wb[...]) + b1[...]
    z = _silu_ln(z)
    z = _dot(z, w2[...]) + b2[...]
    out[...] = _silu_ln(z)


def _merge_e_call(e, t, ps):
    E = e.shape[0]
    w1 = ps[0]['w']
    args = [e, t, w1[:EE], w1[EE:], ps[0]['b'].reshape(1, EE),
            ps[1]['w'], ps[1]['b'].reshape(1, EE)]
    return pl.pallas_call(
        _merge_e_body,
        grid=(E // _EBLK,),
        in_specs=[_rowspec(_EBLK, EE), _rowspec(_EBLK, TD)]
        + [_fs(a.shape) for a in args[2:]],
        out_specs=_rowspec(_EBLK, EE),
        out_shape=jax.ShapeDtypeStruct((E, EE), jnp.float32),
        interpret=_INTERP,
    )(*args)


def _rec_init_body(h_ref, w_ref, out):
    out[...] = _dot(h_ref[...], w_ref[...])


def _rec_init_call(rec_h, w):
    return pl.pallas_call(
        _rec_init_body,
        grid=(N_REC // _NBLK,),
        in_specs=[_rowspec(_NBLK, rec_h.shape[1]), _fs(w.shape)],
        out_specs=_rowspec(_NBLK, H),
        out_shape=jax.ShapeDtypeStruct((N_REC, H), jnp.float32),
        interpret=_INTERP,
    )(rec_h, w)


def _lig_init_body(ty_ref, ba_ref, t_ref, emb, wa, wb, b1, w2, b2,
                   h_out, tb_out):
    blk = ty_ref.shape[0]
    oh = (lax.broadcasted_iota(jnp.int32, (blk, 32), 1)
          == ty_ref[...]).astype(jnp.float32)
    h0 = _dot(oh, emb[...])
    ohb = (lax.broadcasted_iota(jnp.int32, (blk, 8), 1)
           == ba_ref[...]).astype(jnp.float32)
    tb = _dot(ohb, t_ref[...])
    tb_out[...] = tb
    z = _dot(h0, wa[...]) + _dot(tb, wb[...]) + b1[...]
    z = _silu_ln(z)
    z = _dot(z, w2[...]) + b2[...]
    h_out[...] = _silu_ln(z)


def _lig_init_call(lig_h_type, lig_batch, t, emb, ps):
    w1 = ps[0]['w']
    args = [lig_h_type.reshape(N_LIG, 1), lig_batch.reshape(N_LIG, 1), t,
            emb, w1[:H], w1[H:], ps[0]['b'].reshape(1, H),
            ps[1]['w'], ps[1]['b'].reshape(1, H)]
    return pl.pallas_call(
        _lig_init_body,
        grid=(N_LIG // _NBLK,),
        in_specs=[_rowspec(_NBLK, 1), _rowspec(_NBLK, 1)]
        + [_fs(a.shape) for a in args[2:]],
        out_specs=[_rowspec(_NBLK, H), _rowspec(_NBLK, TD)],
        out_shape=[jax.ShapeDtypeStruct((N_LIG, H), jnp.float32),
                   jax.ShapeDtypeStruct((N_LIG, TD), jnp.float32)],
        interpret=_INTERP,
    )(*args)


def _upd_body(nacc, h_ref, wa, wb, b1, w2, b2, *rest):
    accs, out = rest[:nacc], rest[nacc]
    h = h_ref[...]
    agg = accs[0][...][:, :H]
    for a in accs[1:]:
        agg = agg + a[...][:, :H]
    z = _dot(h, wa[...]) + _dot(agg, wb[...]) + b1[...]
    z = _silu_ln(z)
    z = _dot(z, w2[...]) + b2[...]
    out[...] = h + z


def _upd_call(h, accs, ps):
    n = h.shape[0]
    w1 = ps[0]['w']
    flat = []
    for acc in accs:
        for p in range(acc.shape[0]):
            flat.append(acc[p])
    args = [h, w1[:H], w1[H:], ps[0]['b'].reshape(1, HID),
            ps[1]['w'], ps[1]['b'].reshape(1, H)] + flat
    return pl.pallas_call(
        functools.partial(_upd_body, len(flat)),
        grid=(n // _NBLK,),
        in_specs=[_rowspec(_NBLK, H)] + [_fs(a.shape) for a in args[1:6]]
        + [_rowspec(_NBLK, f.shape[1]) for f in flat],
        out_specs=_rowspec(_NBLK, H),
        out_shape=jax.ShapeDtypeStruct((n, H), jnp.float32),
        interpret=_INTERP,
    )(*args)


def _xupd_body(x_ref, *rest):
    accs, out = rest[:-1], rest[-1]
    acc = accs[0][...]
    for a in accs[1:]:
        acc = acc + a[...]
    num = acc[:, HID:HID + XP]
    cnt = acc[:, HID + 3:HID + 4]
    col = lax.broadcasted_iota(jnp.int32, num.shape, 1)
    num = jnp.where(col == 3, 0.0, num)
    out[...] = x_ref[...] + num / (cnt + 1e-8)


def _xupd_call(x, acc):
    n = x.shape[0]
    flat = [acc[p] for p in range(acc.shape[0])]
    return pl.pallas_call(
        _xupd_body,
        grid=(n // _NBLK,),
        in_specs=[_rowspec(_NBLK, XP)]
        + [_rowspec(_NBLK, f.shape[1]) for f in flat],
        out_specs=_rowspec(_NBLK, XP),
        out_shape=jax.ShapeDtypeStruct((n, XP), jnp.float32),
        interpret=_INTERP,
    )(x, *flat)


def _readout_body(h_ref, w1, b1, w2, b2, w3, b3, out):
    z = _dot(h_ref[...], w1[...]) + b1[...]
    z = _silu_ln(z)
    z = _dot(z, w2[...]) + b2[...]
    z = _silu_ln(z)
    out[...] = _dot(z, w3[...]) + b3[...]


def _readout_call(h, ps):
    n = h.shape[0]
    d = ps[2]['w'].shape[1]
    args = [h, ps[0]['w'], ps[0]['b'].reshape(1, -1),
            ps[1]['w'], ps[1]['b'].reshape(1, -1),
            ps[2]['w'], ps[2]['b'].reshape(1, -1)]
    return pl.pallas_call(
        _readout_body,
        grid=(n // _NBLK,),
        in_specs=[_rowspec(_NBLK, H)] + [_fs(a.shape) for a in args[1:]],
        out_specs=_rowspec(_NBLK, d),
        out_shape=jax.ShapeDtypeStruct((n, d), jnp.float32),
        interpret=_INTERP,
    )(*args)


_VMESH = plsc.VectorSubcoreMesh(core_axis_name="c", subcore_axis_name="s")
_W = 80
_NSUB = 16
_NCORE = 2


def _row_share(n, sid):
    per = ((n + _NSUB - 1) // _NSUB + 7) // 8 * 8
    last = n - per * (_NSUB - 1)
    assert last > 0 and last % 8 == 0
    return per, last


def _stage_rows(src, dst, n, sid):
    per, last = _row_share(n, sid)

    @pl.when(sid < _NSUB - 1)
    def _():
        sl = pl.ds(sid * per, per)
        pltpu.sync_copy(src.at[sl], dst.at[sl])

    @pl.when(sid == _NSUB - 1)
    def _():
        sl = pl.ds((_NSUB - 1) * per, last)
        pltpu.sync_copy(src.at[sl], dst.at[sl])


_NW = _NCORE * _NSUB


def _widx(idx):
    nwin = idx.shape[0] // _W
    nk = -(-nwin // _NW)
    idx2 = idx.reshape(nwin, _W)
    idx2 = jnp.pad(idx2, ((0, nk * _NW - nwin), (0, 0)))
    return idx2.reshape(nk, _NW, _W).transpose(1, 0, 2), nwin, nk


def _pipe2(n, start1, wait1, start2, wait2):
    if n > 0:
        start1(0, 0)
    if n > 1:
        start1(1, 1)

    def halfstep(k, b):
        wait1(k, b)
        start2(k, b)

        @pl.when(k + 2 < n)
        def _():
            wait2(k, b)
            start1(k + 2, b)

    @pl.loop(0, n // 2)
    def _(p):
        halfstep(2 * p, 0)
        halfstep(2 * p + 1, 1)

    if n % 2 == 1:
        if n >= 2:
            wait2(n - 2, 1)
        wait1(n - 1, 0)
        start2(n - 1, 0)
        wait2(n - 1, 0)
    else:
        if n >= 2:
            wait2(n - 2, 0)
        if n >= 1:
            wait2(n - 1, 1)


def _gather_rows(table, idx):
    n, D = table.shape
    E = idx.shape[0]
    idx3, nwin, nk = _widx(idx)
    n_main = nwin // _NW
    rem = nwin % _NW

    @functools.partial(
        pl.kernel,
        out_type=jax.ShapeDtypeStruct((E, D), jnp.float32),
        mesh=_VMESH,
        compiler_params=pltpu.CompilerParams(use_tc_tiling_on_sc=False),
        scratch_types=[pltpu.VMEM((nk, _W), jnp.int32),
                       pltpu.VMEM((_W, D), jnp.float32),
                       pltpu.VMEM((_W, D), jnp.float32),
                       pltpu.VMEM_SHARED((n, D), jnp.float32),
                       pltpu.SemaphoreType.DMA,
                       pltpu.SemaphoreType.DMA,
                       pltpu.SemaphoreType.DMA,
                       pltpu.SemaphoreType.DMA],
        name=f"sc_gather_{E}_{D}",
    )
    def k(table_hbm, idx_hbm, out_hbm, i_all, rows_a, rows_b, tab_sh,
          s1a, s1b, s2a, s2b):
        cid = lax.axis_index("c")
        sid = lax.axis_index("s")
        wid = sid * _NCORE + cid
        bufs = (rows_a, rows_b)
        s1 = (s1a, s1b)
        s2 = (s2a, s2b)

        pltpu.sync_copy(idx_hbm.at[wid], i_all)
        _stage_rows(table_hbm, tab_sh, n, sid)
        plsc.subcore_barrier()

        def start1(kk, b):
            pltpu.async_copy(tab_sh.at[i_all.at[kk]], bufs[b], s1[b])

        def wait1(kk, b):
            pltpu.make_async_copy(tab_sh.at[i_all.at[kk]], bufs[b],
                                  s1[b]).wait()

        def _out(kk):
            return out_hbm.at[pl.ds((kk * _NW + wid) * _W, _W)]

        def start2(kk, b):
            pltpu.async_copy(bufs[b], _out(kk), s2[b])

        def wait2(kk, b):
            pltpu.make_async_copy(bufs[b], _out(kk), s2[b]).wait()

        _pipe2(n_main, start1, wait1, start2, wait2)

        if rem:
            @pl.when(wid < rem)
            def _():
                pltpu.sync_copy(tab_sh.at[i_all.at[n_main]], rows_a)
                pltpu.sync_copy(rows_a, _out(n_main))

    return k(table, idx3)


def _scatter_add(pairs, n):
    D = pairs[0][0].shape[1]
    flat = []
    meta = []
    nk_max = 0
    for v, idx in pairs:
        idx3, nwin, nk = _widx(idx)
        flat += [v, idx3]
        meta.append((nwin // _NW, nwin % _NW, nk))
        nk_max = max(nk_max, nk)
    zeros = jnp.zeros((n, D), jnp.float32)
    tag = "_".join(str(v.shape[0]) for v, _ in pairs)

    @functools.partial(
        pl.kernel,
        out_type=jax.ShapeDtypeStruct((_NCORE, n, D), jnp.float32),
        mesh=_VMESH,
        compiler_params=pltpu.CompilerParams(use_tc_tiling_on_sc=False),
        scratch_types=[pltpu.VMEM((nk_max, _W), jnp.int32),
                       pltpu.VMEM((_W, D), jnp.float32),
                       pltpu.VMEM((_W, D), jnp.float32),
                       pltpu.VMEM_SHARED((n, D), jnp.float32),
                       pltpu.SemaphoreType.DMA,
                       pltpu.SemaphoreType.DMA,
                       pltpu.SemaphoreType.DMA,
                       pltpu.SemaphoreType.DMA],
        name=f"sc_scatter_{tag}_{D}",
    )
    def k(*refs):
        *ins, out_hbm, i_all, v_a, v_b, acc_sh, s1a, s1b, s2a, s2b = refs
        zeros_hbm = ins[-1]
        cid = lax.axis_index("c")
        sid = lax.axis_index("s")
        wid = sid * _NCORE + cid
        bufs = (v_a, v_b)
        s1 = (s1a, s1b)
        s2 = (s2a, s2b)

        _stage_rows(zeros_hbm, acc_sh, n, sid)
        plsc.subcore_barrier()

        for p, (n_main, rem, nk) in enumerate(meta):
            vals_hbm = ins[2 * p]
            idx_hbm = ins[2 * p + 1]
            pltpu.sync_copy(idx_hbm.at[wid], i_all.at[pl.ds(0, nk)])

            def _src(kk, vals_hbm=vals_hbm):
                return vals_hbm.at[pl.ds((kk * _NW + wid) * _W, _W)]

            def start1(kk, b):
                pltpu.async_copy(_src(kk), bufs[b], s1[b])

            def wait1(kk, b):
                pltpu.make_async_copy(_src(kk), bufs[b], s1[b]).wait()

            def start2(kk, b):
                pltpu.async_copy(bufs[b], acc_sh.at[i_all.at[kk]], s2[b],
                                 add=True)

            def wait2(kk, b):
                pltpu.make_async_copy(bufs[b], acc_sh.at[i_all.at[kk]],
                                      s2[b]).wait()

            _pipe2(n_main, start1, wait1, start2, wait2)

            if rem:
                @pl.when(wid < rem)
                def _():
                    pltpu.sync_copy(_src(n_main), v_a)
                    pltpu.sync_copy(v_a, acc_sh.at[i_all.at[n_main]],
                                    add=True)

        plsc.subcore_barrier()
        _stage_rows(acc_sh, out_hbm.at[cid], n, sid)

    return k(*flat, zeros)


def kernel(rec_h, rec_x, rec_e_index, rec_e_type, rec_batch, lig_h_type,
           lig_x, lig_e_index, lig_e_type, lig_batch, timestep,
           inter_e_index, inter_e_type, params):
    t = jnp.asarray(_PE)[timestep]
    for i, p in enumerate(params['embd_timestep']):
        t = t @ p['w'] + p['b']
        t = jax.nn.silu(_ln(t))

    r0, r1 = rec_e_index[0], rec_e_index[1]
    l0, l1 = lig_e_index[0], lig_e_index[1]
    isrc, idst = inter_e_index[0], inter_e_index[1]

    rec_xp = jnp.pad(rec_x, ((0, 0), (0, XP - 3)))
    lig_xp = jnp.pad(lig_x, ((0, 0), (0, XP - 3)))

    rec_h_cur = _rec_init_call(rec_h, params['embd_rec_h'])
    lig_h, tB = _lig_init_call(lig_h_type, lig_batch, t,
                               params['embd_lig_h'], params['merge_lig_h'])

    rec_e_raw = _onehot_embed(rec_e_type, params['embd_rec_e'])
    re = [rec_e_raw, rec_e_raw]
    lig_e_raw = _onehot_embed(lig_e_type, params['embd_lig_e'])
    inter_e_raw = _onehot_embed(inter_e_type, params['embd_inter_e'])
    tb0 = _gather_rows(tB, l0)
    tb1 = _gather_rows(tB, l1)
    le = [_merge_e_call(lig_e_raw, tb1, params['merge_lig_e']),
          _merge_e_call(lig_e_raw, tb0, params['merge_lig_e'])]
    t_inter_e = _gather_rows(tB, idst)
    inter_e = _merge_e_call(inter_e_raw, t_inter_e, params['merge_inter_e'])

    for lp in params['layers']:
        tab_r = jnp.concatenate([rec_h_cur, rec_xp], axis=1)
        g0 = _gather_rows(tab_r, r0)
        g1 = _gather_rows(tab_r, r1)
        m0, re0 = _msg_call(g0, g1, re[0], lp['rr_msg'], lp['rr_e'])
        m1, re1 = _msg_call(g1, g0, re[1], lp['rr_msg'], lp['rr_e'])
        re = [re0, re1]
        acc_rr = _scatter_add([(m0, r1), (m1, r0)], N_REC)
        rec_h_cur = _upd_call(rec_h_cur, [acc_rr], lp['rr_upd'])

        tab_l = jnp.concatenate([lig_h, lig_xp], axis=1)
        g0 = _gather_rows(tab_l, l0)
        g1 = _gather_rows(tab_l, l1)
        msc0, le0 = _msg_call(g0, g1, le[0], lp['ll_msg'], lp['ll_e'],
                              lp['ll_x'])
        msc1, le1 = _msg_call(g1, g0, le[1], lp['ll_msg'], lp['ll_e'],
                              lp['ll_x'])
        le = [le0, le1]
        acc_ll = _scatter_add([(msc0, l1), (msc1, l0)], N_LIG)
        lig_xp = _xupd_call(lig_xp, acc_ll)

        tab_r2 = jnp.concatenate([rec_h_cur, rec_xp], axis=1)
        tab_l2 = jnp.concatenate([lig_h, lig_xp], axis=1)
        gr = _gather_rows(tab_r2, isrc)
        gl = _gather_rows(tab_l2, idst)
        msc_rl, inter_e = _msg_call(gr, gl, inter_e,
                                    lp['rl_msg'], lp['rl_e'], lp['rl_x'])
        acc_rl = _scatter_add([(msc_rl, idst)], N_LIG)
        lig_xp = _xupd_call(lig_xp, acc_rl)
        lig_h = _upd_call(lig_h, [acc_ll, acc_rl], lp['ll_upd'])

    lig_h_out = _readout_call(lig_h, params['readout_lig_h'])
    return lig_h_out, lig_xp[:, :3]
